# baseline (speedup 1.0000x reference)
import jax
import jax.numpy as jnp
from jax.experimental import pallas as pl
from jax.experimental.pallas import tpu as pltpu

N_FINE_, N_MID_, N_COARSE_ = 50000, 7143, 1020
NCONVS_ = 2


def _gelu(x):
    return jax.nn.gelu(x, approximate=False)


def _ln(x, g, b):
    m = jnp.mean(x, axis=-1, keepdims=True)
    v = jnp.var(x, axis=-1, keepdims=True)
    return (x - m) / jnp.sqrt(v + 1e-5) * g + b


def _l2norm(x):
    n = jnp.linalg.norm(x, axis=-1, keepdims=True)
    return x / jnp.maximum(n, 1e-8)


def _mlp(x, p, pre):
    return _gelu(_ln(x @ p[pre + '_W'] + p[pre + '_b'], p[pre + '_g'], p[pre + '_beta']))


def _gcn(x, W, b, ei, ew, n):
    h = x @ W
    sl = jnp.arange(n)
    row = jnp.concatenate([ei[0], sl])
    col = jnp.concatenate([ei[1], sl])
    w = jnp.concatenate([ew, jnp.ones((n,), x.dtype)])
    deg = jnp.zeros((n,), x.dtype).at[col].add(w)
    dinv = jax.lax.rsqrt(deg)
    norm = dinv[row] * w * dinv[col]
    out = jnp.zeros((n, h.shape[1]), x.dtype).at[col].add(norm[:, None] * h[row])
    return out + b


def _encoder(x, ei, ew, p, pre, n):
    identity = _mlp(x, p, pre + '_res')
    out = x
    for i in range(NCONVS_):
        out = _gcn(out, p[pre + '_convW' + str(i)], p[pre + '_convb' + str(i)], ei, ew, n)
        out = _gelu(_ln(out, p[pre + '_ng' + str(i)], p[pre + '_nb' + str(i)]))
    return out + identity


def _decoder(x, skip, ei, ew, p, pre, n, proj):
    combined = x + skip
    identity = _mlp(combined, p, pre + '_res')
    out = combined
    for i in range(NCONVS_):
        out = _gcn(out, p[pre + '_convW' + str(i)], p[pre + '_convb' + str(i)], ei, ew, n)
        out = _gelu(_ln(out, p[pre + '_ng' + str(i)], p[pre + '_nb' + str(i)]))
    if proj:
        out = _ln(out @ p[pre + '_p_W'] + p[pre + '_p_b'], p[pre + '_p_g'], p[pre + '_p_beta'])
    return out + identity


def _spmap(x, idx, val, T, p, pre):
    m = jnp.zeros((T, x.shape[1]), x.dtype).at[idx[0]].add(val[:, None] * x[idx[1]])
    return _mlp(m, p, pre)


def _head_body(x_ref, w_ref, b_ref, o_ref):
    o_ref[...] = x_ref[...] @ w_ref[...] + b_ref[...]


def _head(x, W, b):
    n, d = x.shape
    dout = W.shape[1]
    blk = 1000
    grid = (pl.cdiv(n, blk),)
    return pl.pallas_call(
        _head_body,
        grid=grid,
        in_specs=[
            pl.BlockSpec((blk, d), lambda i: (i, 0)),
            pl.BlockSpec((d, dout), lambda i: (0, 0)),
            pl.BlockSpec((1, dout), lambda i: (0, 0)),
        ],
        out_specs=pl.BlockSpec((blk, dout), lambda i: (i, 0)),
        out_shape=jax.ShapeDtypeStruct((n, dout), x.dtype),
    )(x, W, b.reshape(1, -1))


def kernel(features, edge_index_fine, edge_weight_fine, edge_index_mid, edge_weight_mid, edge_index_coarse, edge_weight_coarse, map_f2m_idx, map_f2m_val, map_m2c_idx, map_m2c_val, map_c2m_idx, map_c2m_val, map_m2f_idx, map_m2f_val, params):
    p = params
    x = jnp.nan_to_num(features)
    h = _mlp(x, p, 'fus')
    h = _l2norm(h)
    w = jax.nn.softmax(p['fus_mw'])
    fused = _l2norm(h * w[0])
    e1 = _encoder(fused, edge_index_fine, edge_weight_fine, p, 'enc1', N_FINE_)
    m1 = _spmap(e1, map_f2m_idx, map_f2m_val, N_MID_, p, 'f2m')
    e2 = _encoder(m1, edge_index_mid, edge_weight_mid, p, 'enc2', N_MID_)
    m2 = _spmap(e2, map_m2c_idx, map_m2c_val, N_COARSE_, p, 'm2c')
    e3 = _encoder(m2, edge_index_coarse, edge_weight_coarse, p, 'enc3', N_COARSE_)
    d3 = _decoder(e3, m2, edge_index_coarse, edge_weight_coarse, p, 'dec3', N_COARSE_, True)
    u2 = _spmap(d3, map_c2m_idx, map_c2m_val, N_MID_, p, 'c2m')
    d2 = _decoder(u2, e2, edge_index_mid, edge_weight_mid, p, 'dec2', N_MID_, True)
    u1 = _spmap(d2, map_m2f_idx, map_m2f_val, N_FINE_, p, 'm2f')
    d1 = _decoder(u1, e1, edge_index_fine, edge_weight_fine, p, 'dec1', N_FINE_, False)
    emb_f = _head(d1, p['head_f_W'], p['head_f_b'])
    emb_m = _head(d2, p['head_m_W'], p['head_m_b'])
    emb_c = _head(d3, p['head_c_W'], p['head_c_b'])
    recon = _head(d1, p['rec_W'], p['rec_b'])
    return emb_f, emb_m, emb_c, recon


# speedup vs baseline: 5.9040x; 5.9040x over previous
import functools

import jax
import jax.numpy as jnp
from jax import lax
from jax.experimental import pallas as pl
from jax.experimental.pallas import tpu as pltpu
from jax.experimental.pallas import tpu_sc as plsc

F32 = jnp.float32
I32 = jnp.int32

NFINE, NMID, NCOARSE = 50000, 7143, 1020
NMIDP, NCOARSEP = 7168, 1024
DIN = 208

_MESH = dict(core_axis_name="c", subcore_axis_name="s", num_cores=2,
             num_subcores=16)


def _bs(n):
    return 1000 if n == NFINE else 1024


_BCAST_DNUMS = lax.GatherDimensionNumbers(
    offset_dims=(), collapsed_slice_dims=(0,), start_index_map=(0,))


def _bcast16(vec, j):
    return lax.gather(vec, jnp.full((16, 1), j, I32), _BCAST_DNUMS, (1,),
                      mode=lax.GatherScatterMode.PROMISE_IN_BOUNDS)


def _gelu(x):
    return 0.5 * x * (1.0 + lax.erf(x * 0.7071067811865476))


def _lnb(x, g, b):
    m = jnp.mean(x, axis=-1, keepdims=True)
    v = jnp.mean((x - m) ** 2, axis=-1, keepdims=True)
    return (x - m) / jnp.sqrt(v + 1e-5) * g + b


def _pad1(a, m):
    e = a.shape[0]
    ep = -(-e // m) * m
    if ep == e:
        return a
    return jnp.pad(a, ((0, ep - e),))


def _sc_scatter(h, row, col, val, n_pad, d):
    e = row.shape[0]
    k = 128 if d <= 128 else 64
    assert e % (k * 32) == 0, (e, k)
    cpw = e // (k * 32)
    tot, rem = divmod(n_pad, k)
    nloop = -(-tot // 16)
    rem_tile = tot % 16

    @functools.partial(
        pl.kernel,
        out_type=jax.ShapeDtypeStruct((2 * n_pad, d), F32),
        mesh=plsc.VectorSubcoreMesh(**_MESH),
        compiler_params=pltpu.CompilerParams(use_tc_tiling_on_sc=False),
        scratch_types=[
            pltpu.VMEM((k,), I32),
            pltpu.VMEM((k,), I32),
            pltpu.VMEM((k,), F32),
            pltpu.VMEM((k, d), F32),
            pltpu.VMEM_SHARED((n_pad, d), F32),
        ],
    )
    def kern(h_h, row_h, col_h, val_h, out_h, rbuf, cbuf, vbuf, rows, acc):
        cid = lax.axis_index("c")
        sid = lax.axis_index("s")
        wid = cid * 16 + sid
        zz = jnp.zeros((16,), F32)
        for j in range(k):
            for c0 in range(0, d, 16):
                rows.at[j][pl.ds(c0, 16)] = zz

        def zbody(i, cy):
            c = i * 16 + sid

            @pl.when(c < tot)
            def _():
                pltpu.sync_copy(rows, acc.at[pl.ds(pl.multiple_of(c * k, k), k)])
            return cy

        lax.fori_loop(0, nloop, zbody, 0)
        if rem:
            @pl.when(sid == rem_tile)
            def _():
                pltpu.sync_copy(rows.at[pl.ds(0, rem)],
                                acc.at[pl.ds(tot * k, rem)])
        plsc.subcore_barrier()

        def chunk(i, cy):
            base = pl.multiple_of((wid * cpw + i) * k, k)
            pltpu.sync_copy(row_h.at[pl.ds(base, k)], rbuf)
            pltpu.sync_copy(col_h.at[pl.ds(base, k)], cbuf)
            pltpu.sync_copy(val_h.at[pl.ds(base, k)], vbuf)
            pltpu.sync_copy(h_h.at[rbuf], rows)
            for j0 in range(0, k, 16):
                vgrp = vbuf[pl.ds(j0, 16)]
                for jj in range(16):
                    bv = _bcast16(vgrp, jj)
                    for c0 in range(0, d, 16):
                        rows.at[j0 + jj][pl.ds(c0, 16)] = (
                            rows[j0 + jj, pl.ds(c0, 16)] * bv)
            pltpu.sync_copy(rows, acc.at[cbuf], add=True)
            return cy

        lax.fori_loop(0, cpw, chunk, 0)
        plsc.subcore_barrier()

        def wbody(i, cy):
            c = i * 16 + sid

            @pl.when(c < tot)
            def _():
                r0 = pl.multiple_of(c * k, k)
                pltpu.sync_copy(acc.at[pl.ds(r0, k)], rows)
                pltpu.sync_copy(
                    rows, out_h.at[pl.ds(pl.multiple_of(cid * n_pad + r0, 8), k)])
            return cy

        lax.fori_loop(0, nloop, wbody, 0)
        if rem:
            @pl.when(sid == rem_tile)
            def _():
                pltpu.sync_copy(acc.at[pl.ds(tot * k, rem)],
                                rows.at[pl.ds(0, rem)])
                pltpu.sync_copy(
                    rows.at[pl.ds(0, rem)],
                    out_h.at[pl.ds(pl.multiple_of(cid * n_pad + tot * k, 8), rem)])

    return kern(h, row, col, val)


def _sc_deg(col, val, n_pad):
    e = col.shape[0]
    k = 128
    d = 16
    assert e % (k * 32) == 0
    cpw = e // (k * 32)
    tot, rem = divmod(n_pad, k)
    nloop = -(-tot // 16)
    rem_tile = tot % 16

    @functools.partial(
        pl.kernel,
        out_type=jax.ShapeDtypeStruct((2 * n_pad, d), F32),
        mesh=plsc.VectorSubcoreMesh(**_MESH),
        compiler_params=pltpu.CompilerParams(use_tc_tiling_on_sc=False),
        scratch_types=[
            pltpu.VMEM((k,), I32),
            pltpu.VMEM((k,), F32),
            pltpu.VMEM((k, d), F32),
            pltpu.VMEM_SHARED((n_pad, d), F32),
        ],
    )
    def kern(col_h, val_h, out_h, cbuf, vbuf, rows, acc):
        cid = lax.axis_index("c")
        sid = lax.axis_index("s")
        wid = cid * 16 + sid
        zz = jnp.zeros((16,), F32)
        for j in range(k):
            rows.at[j][pl.ds(0, 16)] = zz

        def zbody(i, cy):
            c = i * 16 + sid

            @pl.when(c < tot)
            def _():
                pltpu.sync_copy(rows, acc.at[pl.ds(pl.multiple_of(c * k, k), k)])
            return cy

        lax.fori_loop(0, nloop, zbody, 0)
        if rem:
            @pl.when(sid == rem_tile)
            def _():
                pltpu.sync_copy(rows.at[pl.ds(0, rem)],
                                acc.at[pl.ds(tot * k, rem)])
        plsc.subcore_barrier()

        def chunk(i, cy):
            base = pl.multiple_of((wid * cpw + i) * k, k)
            pltpu.sync_copy(col_h.at[pl.ds(base, k)], cbuf)
            pltpu.sync_copy(val_h.at[pl.ds(base, k)], vbuf)
            for j0 in range(0, k, 16):
                vgrp = vbuf[pl.ds(j0, 16)]
                for jj in range(16):
                    rows.at[j0 + jj][pl.ds(0, 16)] = _bcast16(vgrp, jj)
            pltpu.sync_copy(rows, acc.at[cbuf], add=True)
            return cy

        lax.fori_loop(0, cpw, chunk, 0)
        plsc.subcore_barrier()

        def wbody(i, cy):
            c = i * 16 + sid

            @pl.when(c < tot)
            def _():
                r0 = pl.multiple_of(c * k, k)
                pltpu.sync_copy(acc.at[pl.ds(r0, k)], rows)
                pltpu.sync_copy(
                    rows, out_h.at[pl.ds(pl.multiple_of(cid * n_pad + r0, 8), k)])
            return cy

        lax.fori_loop(0, nloop, wbody, 0)
        if rem:
            @pl.when(sid == rem_tile)
            def _():
                pltpu.sync_copy(acc.at[pl.ds(tot * k, rem)],
                                rows.at[pl.ds(0, rem)])
                pltpu.sync_copy(
                    rows.at[pl.ds(0, rem)],
                    out_h.at[pl.ds(pl.multiple_of(cid * n_pad + tot * k, 8), rem)])

    return kern(col, val)


def _sc_norm(row, col, w, dinv):
    e = row.shape[0]
    k = 128
    assert e % (k * 32) == 0
    cpw = e // (k * 32)

    @functools.partial(
        pl.kernel,
        out_type=jax.ShapeDtypeStruct((e,), F32),
        mesh=plsc.VectorSubcoreMesh(**_MESH),
        compiler_params=pltpu.CompilerParams(use_tc_tiling_on_sc=False),
        scratch_types=[
            pltpu.VMEM((k,), I32),
            pltpu.VMEM((k,), I32),
            pltpu.VMEM((k,), F32),
            pltpu.VMEM((k,), F32),
            pltpu.VMEM((k,), F32),
            pltpu.VMEM((k,), F32),
        ],
    )
    def kern(row_h, col_h, w_h, dinv_h, out_h, rbuf, cbuf, vbuf, nrb, ncb,
             obuf):
        cid = lax.axis_index("c")
        sid = lax.axis_index("s")
        wid = cid * 16 + sid

        def chunk(i, cy):
            base = pl.multiple_of((wid * cpw + i) * k, k)
            pltpu.sync_copy(row_h.at[pl.ds(base, k)], rbuf)
            pltpu.sync_copy(col_h.at[pl.ds(base, k)], cbuf)
            pltpu.sync_copy(w_h.at[pl.ds(base, k)], vbuf)
            pltpu.sync_copy(dinv_h.at[rbuf], nrb)
            pltpu.sync_copy(dinv_h.at[cbuf], ncb)
            for j0 in range(0, k, 16):
                sl = pl.ds(j0, 16)
                obuf[sl] = nrb[sl] * vbuf[sl] * ncb[sl]
            pltpu.sync_copy(obuf, out_h.at[pl.ds(base, k)])
            return cy

        lax.fori_loop(0, cpw, chunk, 0)

    return kern(row, col, w, dinv)


def _r2(a):
    return a.reshape(1, -1)


def _tc_fus(x, w, b, g, be, mw):
    n, din = x.shape
    dout = w.shape[1]
    blk = _bs(n)

    def body(x_r, w_r, b_r, g_r, be_r, mw_r, o_r):
        xx = jnp.nan_to_num(x_r[...])
        h = jnp.dot(xx, w_r[...], preferred_element_type=F32) + b_r[...]
        h = _gelu(_lnb(h, g_r[...], be_r[...]))
        nn = jnp.sqrt(jnp.sum(h * h, -1, keepdims=True))
        h = h / jnp.maximum(nn, 1e-8)
        m = mw_r[0, 0]
        h = h * jnp.exp(m - m)
        nn = jnp.sqrt(jnp.sum(h * h, -1, keepdims=True))
        o_r[...] = h / jnp.maximum(nn, 1e-8)

    return pl.pallas_call(
        body,
        grid=(n // blk,),
        in_specs=[
            pl.BlockSpec((blk, din), lambda i: (i, 0)),
            pl.BlockSpec((din, dout), lambda i: (0, 0)),
            pl.BlockSpec((1, dout), lambda i: (0, 0)),
            pl.BlockSpec((1, dout), lambda i: (0, 0)),
            pl.BlockSpec((1, dout), lambda i: (0, 0)),
            pl.BlockSpec((1, 1), lambda i: (0, 0)),
        ],
        out_specs=pl.BlockSpec((blk, dout), lambda i: (i, 0)),
        out_shape=jax.ShapeDtypeStruct((n, dout), F32),
    )(x, w, _r2(b), _r2(g), _r2(be), mw.reshape(1, 1))


def _tc_mm(x, w, skip=None, halves=False):
    n, din = x.shape
    dout = w.shape[1]
    blk = _bs(n)
    dh = dout // 2

    def body(*refs):
        if skip is not None:
            x_r, s_r, w_r = refs[0], refs[1], refs[2]
            outs = refs[3:]
            xx = x_r[...] + s_r[...]
        else:
            x_r, w_r = refs[0], refs[1]
            outs = refs[2:]
            xx = x_r[...]
        h = jnp.dot(xx, w_r[...], preferred_element_type=F32)
        if halves:
            outs[0][...] = h[:, :dh]
            outs[1][...] = h[:, dh:]
        else:
            outs[0][...] = h

    in_specs = [pl.BlockSpec((blk, din), lambda i: (i, 0))]
    args = [x]
    if skip is not None:
        in_specs.append(pl.BlockSpec((blk, din), lambda i: (i, 0)))
        args.append(skip)
    in_specs.append(pl.BlockSpec((din, dout), lambda i: (0, 0)))
    args.append(w)
    if halves:
        out_specs = [pl.BlockSpec((blk, dh), lambda i: (i, 0))] * 2
        out_shape = [jax.ShapeDtypeStruct((n, dh), F32)] * 2
    else:
        out_specs = pl.BlockSpec((blk, dout), lambda i: (i, 0))
        out_shape = jax.ShapeDtypeStruct((n, dout), F32)
    return pl.pallas_call(body, grid=(n // blk,), in_specs=in_specs,
                          out_specs=out_specs, out_shape=out_shape)(*args)


def _tc_post(parts, hs, dinv2, bias, ng, nb, idn=None):
    n2, dh = parts[0].shape
    n = n2 // 2
    d = dh * len(parts)
    blk = _bs(n)
    nblk = n // blk
    np_ = len(parts)
    nh = len(hs)

    def body(*refs):
        prefs = refs[:2 * np_]
        hrefs = refs[2 * np_:2 * np_ + nh]
        dv_r, b_r, g_r, nb_r = refs[2 * np_ + nh:2 * np_ + nh + 4]
        rest = refs[2 * np_ + nh + 4:]
        pieces = [prefs[2 * t][...] + prefs[2 * t + 1][...] for t in range(np_)]
        s = pieces[0] if np_ == 1 else jnp.concatenate(pieces, axis=-1)
        hvals = [r[...] for r in hrefs]
        h = hvals[0] if nh == 1 else jnp.concatenate(hvals, axis=-1)
        dv = dv_r[...][:, 0:1]
        y = _gelu(_lnb(s + h * (dv * dv) + b_r[...], g_r[...], nb_r[...]))
        if idn is not None:
            y = y + rest[0][...]
        rest[-1][...] = y

    in_specs = []
    args = []
    for pt in parts:
        in_specs.append(pl.BlockSpec((blk, dh), lambda i: (i, 0)))
        args.append(pt)
        in_specs.append(pl.BlockSpec((blk, dh), lambda i, nb_=nblk: (i + nb_, 0)))
        args.append(pt)
    for hv in hs:
        in_specs.append(pl.BlockSpec((blk, dh), lambda i: (i, 0)))
        args.append(hv)
    in_specs.append(pl.BlockSpec((blk, 16), lambda i: (i, 0)))
    args.append(dinv2)
    for v in (bias, ng, nb):
        in_specs.append(pl.BlockSpec((1, d), lambda i: (0, 0)))
        args.append(_r2(v))
    if idn is not None:
        in_specs.append(pl.BlockSpec((blk, d), lambda i: (i, 0)))
        args.append(idn)
    return pl.pallas_call(
        body, grid=(nblk,), in_specs=in_specs,
        out_specs=pl.BlockSpec((blk, d), lambda i: (i, 0)),
        out_shape=jax.ShapeDtypeStruct((n, d), F32),
    )(*args)


def _tc_mlp(xs, w, b, g, be, mode):
    if mode in ('plain', 'skip'):
        n, din = xs[0].shape
    elif mode == 'part':
        n = xs[0].shape[0] // 2
        din = xs[0].shape[1]
    else:
        n = xs[0].shape[0] // 2
        din = xs[0].shape[1] * 2
    dout = w.shape[1]
    blk = _bs(n)
    nblk = n // blk

    def body(*refs):
        if mode == 'plain':
            xx = refs[0][...]
            k0 = 1
        elif mode == 'skip':
            xx = refs[0][...] + refs[1][...]
            k0 = 2
        elif mode == 'part':
            xx = refs[0][...] + refs[1][...]
            k0 = 2
        else:
            xx = jnp.concatenate(
                [refs[0][...] + refs[1][...], refs[2][...] + refs[3][...]],
                axis=-1)
            k0 = 4
        w_r, b_r, g_r, be_r, o_r = refs[k0:k0 + 5]
        h = jnp.dot(xx, w_r[...], preferred_element_type=F32) + b_r[...]
        o_r[...] = _gelu(_lnb(h, g_r[...], be_r[...]))

    in_specs = []
    args = []
    if mode == 'plain':
        in_specs.append(pl.BlockSpec((blk, din), lambda i: (i, 0)))
        args.append(xs[0])
    elif mode == 'skip':
        for a in xs:
            in_specs.append(pl.BlockSpec((blk, din), lambda i: (i, 0)))
            args.append(a)
    elif mode == 'part':
        in_specs.append(pl.BlockSpec((blk, din), lambda i: (i, 0)))
        args.append(xs[0])
        in_specs.append(pl.BlockSpec((blk, din), lambda i, nb_=nblk: (i + nb_, 0)))
        args.append(xs[0])
    else:
        dh = din // 2
        for a in xs:
            in_specs.append(pl.BlockSpec((blk, dh), lambda i: (i, 0)))
            args.append(a)
            in_specs.append(pl.BlockSpec((blk, dh), lambda i, nb_=nblk: (i + nb_, 0)))
            args.append(a)
    in_specs.append(pl.BlockSpec((din, dout), lambda i: (0, 0)))
    args.append(w)
    for v in (b, g, be):
        in_specs.append(pl.BlockSpec((1, dout), lambda i: (0, 0)))
        args.append(_r2(v))
    return pl.pallas_call(
        body, grid=(nblk,), in_specs=in_specs,
        out_specs=pl.BlockSpec((blk, dout), lambda i: (i, 0)),
        out_shape=jax.ShapeDtypeStruct((n, dout), F32),
    )(*args)


def _tc_proj(y, idn, w, b, g, be, halves=False):
    n, din = y.shape
    dout = w.shape[1]
    blk = _bs(n)
    dh = dout // 2

    def body(y_r, id_r, w_r, b_r, g_r, be_r, *outs):
        h = jnp.dot(y_r[...], w_r[...], preferred_element_type=F32) + b_r[...]
        out = _lnb(h, g_r[...], be_r[...]) + id_r[...]
        outs[0][...] = out
        if halves:
            outs[1][...] = out[:, :dh]
            outs[2][...] = out[:, dh:]

    out_specs = [pl.BlockSpec((blk, dout), lambda i: (i, 0))]
    out_shape = [jax.ShapeDtypeStruct((n, dout), F32)]
    if halves:
        out_specs += [pl.BlockSpec((blk, dh), lambda i: (i, 0))] * 2
        out_shape += [jax.ShapeDtypeStruct((n, dh), F32)] * 2
    res = pl.pallas_call(
        body, grid=(n // blk,),
        in_specs=[
            pl.BlockSpec((blk, din), lambda i: (i, 0)),
            pl.BlockSpec((blk, dout), lambda i: (i, 0)),
            pl.BlockSpec((din, dout), lambda i: (0, 0)),
            pl.BlockSpec((1, dout), lambda i: (0, 0)),
            pl.BlockSpec((1, dout), lambda i: (0, 0)),
            pl.BlockSpec((1, dout), lambda i: (0, 0)),
        ],
        out_specs=out_specs if halves else out_specs[0],
        out_shape=out_shape if halves else out_shape[0],
    )(y, idn, w, _r2(b), _r2(g), _r2(be))
    return res


def _tc_dinv(degp, n_pad):
    f = n_pad * 16
    x = degp.reshape(2, f)
    blk = 6400 if f % 6400 == 0 else 4096

    def body(p_r, o_r):
        xx = p_r[...]
        o_r[...] = lax.rsqrt(1.0 + xx[0:1, :] + xx[1:2, :])

    out = pl.pallas_call(
        body, grid=(f // blk,),
        in_specs=[pl.BlockSpec((2, blk), lambda i: (0, i))],
        out_specs=pl.BlockSpec((1, blk), lambda i: (0, i)),
        out_shape=jax.ShapeDtypeStruct((1, f), F32),
    )(x)
    return out.reshape(n_pad, 16)


def _tc_head(x, w, b):
    n, din = x.shape
    dout = w.shape[1]
    blk = _bs(n)

    def body(x_r, w_r, b_r, o_r):
        o_r[...] = jnp.dot(x_r[...], w_r[...],
                           preferred_element_type=F32) + b_r[...]

    return pl.pallas_call(
        body, grid=(n // blk,),
        in_specs=[
            pl.BlockSpec((blk, din), lambda i: (i, 0)),
            pl.BlockSpec((din, dout), lambda i: (0, 0)),
            pl.BlockSpec((1, dout), lambda i: (0, 0)),
        ],
        out_specs=pl.BlockSpec((blk, dout), lambda i: (i, 0)),
        out_shape=jax.ShapeDtypeStruct((n, dout), F32),
    )(x, w, _r2(b))


def _tc_head2(x, w1, b1, w2, b2):
    n, din = x.shape
    d1, d2 = w1.shape[1], w2.shape[1]
    blk = _bs(n)

    def body(x_r, w1_r, b1_r, w2_r, b2_r, o1_r, o2_r):
        xx = x_r[...]
        o1_r[...] = jnp.dot(xx, w1_r[...], preferred_element_type=F32) + b1_r[...]
        o2_r[...] = jnp.dot(xx, w2_r[...], preferred_element_type=F32) + b2_r[...]

    return pl.pallas_call(
        body, grid=(n // blk,),
        in_specs=[
            pl.BlockSpec((blk, din), lambda i: (i, 0)),
            pl.BlockSpec((din, d1), lambda i: (0, 0)),
            pl.BlockSpec((1, d1), lambda i: (0, 0)),
            pl.BlockSpec((din, d2), lambda i: (0, 0)),
            pl.BlockSpec((1, d2), lambda i: (0, 0)),
        ],
        out_specs=[
            pl.BlockSpec((blk, d1), lambda i: (i, 0)),
            pl.BlockSpec((blk, d2), lambda i: (i, 0)),
        ],
        out_shape=[
            jax.ShapeDtypeStruct((n, d1), F32),
            jax.ShapeDtypeStruct((n, d2), F32),
        ],
    )(x, w1, _r2(b1), w2, _r2(b2))


def _conv_block(x, skip, rowp, colp, normp, dinv2, p, pre, n_pad, d, fine,
                add_idn_last, idn):
    out = x
    for i in range(2):
        w = p[pre + '_convW' + str(i)]
        b = p[pre + '_convb' + str(i)]
        ng = p[pre + '_ng' + str(i)]
        nb = p[pre + '_nb' + str(i)]
        sk = skip if i == 0 else None
        last = add_idn_last and i == 1
        if fine:
            ha, hb = _tc_mm(out, w, skip=sk, halves=True)
            pa = _sc_scatter(ha, rowp, colp, normp, n_pad, d // 2)
            pb = _sc_scatter(hb, rowp, colp, normp, n_pad, d // 2)
            out = _tc_post([pa, pb], [ha, hb], dinv2, b, ng, nb,
                           idn=idn if last else None)
        else:
            h = _tc_mm(out, w, skip=sk)
            pp = _sc_scatter(h, rowp, colp, normp, n_pad, d)
            out = _tc_post([pp], [h], dinv2, b, ng, nb,
                           idn=idn if last else None)
    return out


def _encoder(x, ed, p, pre, n_pad, d, fine):
    idn = _tc_mlp((x,), p[pre + '_res_W'], p[pre + '_res_b'],
                  p[pre + '_res_g'], p[pre + '_res_beta'], 'plain')
    return _conv_block(x, None, ed['row'], ed['col'], ed['norm'], ed['dinv2'],
                       p, pre, n_pad, d, fine, True, idn)


def _decoder(x, skip, ed, p, pre, n_pad, d, fine, proj, halves=False):
    idn = _tc_mlp((x, skip), p[pre + '_res_W'], p[pre + '_res_b'],
                  p[pre + '_res_g'], p[pre + '_res_beta'], 'skip')
    out = _conv_block(x, skip, ed['row'], ed['col'], ed['norm'], ed['dinv2'],
                      p, pre, n_pad, d, fine, not proj, idn)
    if proj:
        return _tc_proj(out, idn, p[pre + '_p_W'], p[pre + '_p_b'],
                        p[pre + '_p_g'], p[pre + '_p_beta'], halves=halves)
    return out


def _edge_level(ei, ew, n, n_pad):
    row = _pad1(ei[0], 4096)
    col = _pad1(ei[1], 4096)
    w = _pad1(ew, 4096)
    degp = _sc_deg(col, w, n_pad)
    dinv2 = _tc_dinv(degp, n_pad)
    dinv1 = dinv2[:, 0]
    norm = _sc_norm(row, col, w, dinv1)
    return dict(row=row, col=col, norm=norm, dinv2=dinv2, dinv1=dinv1)


def kernel(features, edge_index_fine, edge_weight_fine, edge_index_mid,
           edge_weight_mid, edge_index_coarse, edge_weight_coarse,
           map_f2m_idx, map_f2m_val, map_m2c_idx, map_m2c_val, map_c2m_idx,
           map_c2m_val, map_m2f_idx, map_m2f_val, params):
    p = params

    fused = _tc_fus(features, p['fus_W'], p['fus_b'], p['fus_g'],
                    p['fus_beta'], p['fus_mw'])

    edf = _edge_level(edge_index_fine, edge_weight_fine, NFINE, NFINE)
    edm = _edge_level(edge_index_mid, edge_weight_mid, NMID, NMIDP)
    edc = _edge_level(edge_index_coarse, edge_weight_coarse, NCOARSE, NCOARSEP)

    e1 = _encoder(fused, edf, p, 'enc1', NFINE, 64, True)

    f2m_r = _pad1(map_f2m_idx[1], 4096)
    f2m_c = _pad1(map_f2m_idx[0], 4096)
    f2m_v = _pad1(map_f2m_val, 4096)
    m1p = _sc_scatter(e1, f2m_r, f2m_c, f2m_v, NMIDP, 64)
    m1 = _tc_mlp((m1p,), p['f2m_W'], p['f2m_b'], p['f2m_g'], p['f2m_beta'],
                 'part')

    e2 = _encoder(m1, edm, p, 'enc2', NMIDP, 128, False)

    m2c_r = _pad1(map_m2c_idx[1], 4096)
    m2c_c = _pad1(map_m2c_idx[0], 4096)
    m2c_v = _pad1(map_m2c_val, 4096)
    m2p = _sc_scatter(e2, m2c_r, m2c_c, m2c_v, NCOARSEP, 128)
    m2 = _tc_mlp((m2p,), p['m2c_W'], p['m2c_b'], p['m2c_g'], p['m2c_beta'],
                 'part')

    e3 = _encoder(m2, edc, p, 'enc3', NCOARSEP, 256, False)

    d3 = _decoder(e3, m2, edc, p, 'dec3', NCOARSEP, 256, False, True)

    c2m_r = _pad1(map_c2m_idx[1], 4096)
    c2m_c = _pad1(map_c2m_idx[0], 4096)
    c2m_v = _pad1(map_c2m_val, 4096)
    u2p = _sc_scatter(d3, c2m_r, c2m_c, c2m_v, NMIDP, 128)
    u2 = _tc_mlp((u2p,), p['c2m_W'], p['c2m_b'], p['c2m_g'], p['c2m_beta'],
                 'part')

    d2, d2a, d2b = _decoder(u2, e2, edm, p, 'dec2', NMIDP, 128, False, True,
                            halves=True)

    m2f_r = _pad1(map_m2f_idx[1], 4096)
    m2f_c = _pad1(map_m2f_idx[0], 4096)
    m2f_v = _pad1(map_m2f_val, 4096)
    u1pa = _sc_scatter(d2a, m2f_r, m2f_c, m2f_v, NFINE, 32)
    u1pb = _sc_scatter(d2b, m2f_r, m2f_c, m2f_v, NFINE, 32)
    u1 = _tc_mlp((u1pa, u1pb), p['m2f_W'], p['m2f_b'], p['m2f_g'],
                 p['m2f_beta'], 'part2')

    d1 = _decoder(u1, e1, edf, p, 'dec1', NFINE, 64, True, False)

    emb_f, recon = _tc_head2(d1, p['head_f_W'], p['head_f_b'], p['rec_W'],
                             p['rec_b'])
    emb_m = _tc_head(d2, p['head_m_W'], p['head_m_b'])[:NMID]
    emb_c = _tc_head(d3, p['head_c_W'], p['head_c_b'])[:NCOARSE]
    return emb_f, emb_m, emb_c, recon


# speedup vs baseline: 7.1312x; 1.2079x over previous
import functools

import jax
import jax.numpy as jnp
from jax import lax
from jax.experimental import pallas as pl
from jax.experimental.pallas import tpu as pltpu
from jax.experimental.pallas import tpu_sc as plsc

F32 = jnp.float32
I32 = jnp.int32

NFINE, NMID, NCOARSE = 50000, 7143, 1020
NMIDP, NCOARSEP = 7168, 1024
DIN = 208

_MESH = dict(core_axis_name="c", subcore_axis_name="s", num_cores=2,
             num_subcores=16)


def _bs(n):
    return 1000 if n == NFINE else 1024


_BCAST_DNUMS = lax.GatherDimensionNumbers(
    offset_dims=(), collapsed_slice_dims=(0,), start_index_map=(0,))


def _bcast16(vec, j):
    return lax.gather(vec, jnp.full((16, 1), j, I32), _BCAST_DNUMS, (1,),
                      mode=lax.GatherScatterMode.PROMISE_IN_BOUNDS)


def _gelu(x):
    return 0.5 * x * (1.0 + lax.erf(x * 0.7071067811865476))


def _lnb(x, g, b):
    m = jnp.mean(x, axis=-1, keepdims=True)
    v = jnp.mean((x - m) ** 2, axis=-1, keepdims=True)
    return (x - m) / jnp.sqrt(v + 1e-5) * g + b


def _pad1(a, m):
    e = a.shape[0]
    ep = -(-e // m) * m
    if ep == e:
        return a
    return jnp.pad(a, ((0, ep - e),))


def _k_for_d(d):
    if d <= 64:
        return 128
    if d == 128:
        return 64
    return 32


def _sc_scatter(h, row, col, val, n_pad, d):
    e = row.shape[0]
    k = _k_for_d(d)
    assert e % (k * 32) == 0, (e, k)
    cpw = e // (k * 32)
    npairs = cpw // 2
    odd = cpw % 2
    tot, rem = divmod(n_pad, k)
    nloop = -(-tot // 16)
    rem_tile = tot % 16

    @functools.partial(
        pl.kernel,
        out_type=jax.ShapeDtypeStruct((2 * n_pad, d), F32),
        mesh=plsc.VectorSubcoreMesh(**_MESH),
        compiler_params=pltpu.CompilerParams(use_tc_tiling_on_sc=False),
        scratch_types=[
            pltpu.VMEM((k,), I32),
            pltpu.VMEM((k,), I32),
            pltpu.VMEM((k,), I32),
            pltpu.VMEM((k,), I32),
            pltpu.VMEM((k,), F32),
            pltpu.VMEM((k,), F32),
            pltpu.VMEM((k, d), F32),
            pltpu.VMEM((k, d), F32),
            pltpu.VMEM_SHARED((n_pad, d), F32),
            pltpu.SemaphoreType.DMA,
            pltpu.SemaphoreType.DMA,
        ],
    )
    def kern(h_h, row_h, col_h, val_h, out_h, rb0, rb1, cb0, cb1, vb0, vb1,
             rw0, rw1, acc, gs0, gs1):
        cid = lax.axis_index("c")
        sid = lax.axis_index("s")
        wid = cid * 16 + sid
        wbase = wid * cpw

        def scale(rw, vb):
            for j0 in range(0, k, 16):
                vgrp = vb[pl.ds(j0, 16)]
                for jj in range(16):
                    bv = _bcast16(vgrp, jj)
                    for c0 in range(0, d, 16):
                        rw.at[j0 + jj][pl.ds(c0, 16)] = (
                            rw[j0 + jj, pl.ds(c0, 16)] * bv)

        def issue(c, rb, cb, vb, rw, gs):
            base = pl.multiple_of(c * k, k)
            pltpu.sync_copy(row_h.at[pl.ds(base, k)], rb)
            pltpu.sync_copy(col_h.at[pl.ds(base, k)], cb)
            pltpu.sync_copy(val_h.at[pl.ds(base, k)], vb)
            pltpu.async_copy(h_h.at[rb], rw, gs)

        def consume(rb, cb, vb, rw, gs):
            pltpu.make_async_copy(h_h.at[rb], rw, gs).wait()
            scale(rw, vb)
            pltpu.sync_copy(rw, acc.at[cb], add=True)

        zz = jnp.zeros((16,), F32)
        for j in range(k):
            for c0 in range(0, d, 16):
                rw0.at[j][pl.ds(c0, 16)] = zz

        def zbody(i, cy):
            c = i * 16 + sid

            @pl.when(c < tot)
            def _():
                pltpu.sync_copy(rw0, acc.at[pl.ds(pl.multiple_of(c * k, k), k)])
            return cy

        lax.fori_loop(0, nloop, zbody, 0)
        if rem:
            @pl.when(sid == rem_tile)
            def _():
                pltpu.sync_copy(rw0.at[pl.ds(0, rem)],
                                acc.at[pl.ds(tot * k, rem)])
        plsc.subcore_barrier()

        if npairs:
            issue(wbase, rb0, cb0, vb0, rw0, gs0)
            issue(wbase + 1, rb1, cb1, vb1, rw1, gs1)

            def chunk2(i2, cy):
                c0 = wbase + i2 * 2
                consume(rb0, cb0, vb0, rw0, gs0)
                issue(c0 + 2, rb0, cb0, vb0, rw0, gs0)
                consume(rb1, cb1, vb1, rw1, gs1)
                issue(c0 + 3, rb1, cb1, vb1, rw1, gs1)
                return cy

            lax.fori_loop(0, npairs - 1, chunk2, 0)
            consume(rb0, cb0, vb0, rw0, gs0)
            consume(rb1, cb1, vb1, rw1, gs1)
        if odd:
            issue(wbase + cpw - 1, rb0, cb0, vb0, rw0, gs0)
            consume(rb0, cb0, vb0, rw0, gs0)
        plsc.subcore_barrier()

        def wbody(i, cy):
            c = i * 16 + sid

            @pl.when(c < tot)
            def _():
                r0 = pl.multiple_of(c * k, k)
                pltpu.sync_copy(acc.at[pl.ds(r0, k)], rw0)
                pltpu.sync_copy(
                    rw0, out_h.at[pl.ds(pl.multiple_of(cid * n_pad + r0, 8), k)])
            return cy

        lax.fori_loop(0, nloop, wbody, 0)
        if rem:
            @pl.when(sid == rem_tile)
            def _():
                pltpu.sync_copy(acc.at[pl.ds(tot * k, rem)],
                                rw0.at[pl.ds(0, rem)])
                pltpu.sync_copy(
                    rw0.at[pl.ds(0, rem)],
                    out_h.at[pl.ds(pl.multiple_of(cid * n_pad + tot * k, 8), rem)])

    return kern(h, row, col, val)


def _sc_scatter_split(hcat, row, col, val, n_pad, dh, n_src):
    e = row.shape[0]
    d = dh
    k = _k_for_d(d)
    assert e % (k * 16) == 0, (e, k)
    cpw = e // (k * 16)
    npairs = cpw // 2
    odd = cpw % 2
    tot, rem = divmod(n_pad, k)
    nloop = -(-tot // 16)
    rem_tile = tot % 16

    @functools.partial(
        pl.kernel,
        out_type=jax.ShapeDtypeStruct((2 * n_pad, dh), F32),
        mesh=plsc.VectorSubcoreMesh(**_MESH),
        compiler_params=pltpu.CompilerParams(use_tc_tiling_on_sc=False),
        scratch_types=[
            pltpu.VMEM((k,), I32),
            pltpu.VMEM((k,), I32),
            pltpu.VMEM((k,), I32),
            pltpu.VMEM((k,), I32),
            pltpu.VMEM((k,), F32),
            pltpu.VMEM((k,), F32),
            pltpu.VMEM((k, d), F32),
            pltpu.VMEM((k, d), F32),
            pltpu.VMEM_SHARED((n_pad, d), F32),
            pltpu.SemaphoreType.DMA,
            pltpu.SemaphoreType.DMA,
        ],
    )
    def kern(h_h, row_h, col_h, val_h, out_h, rb0, rb1, cb0, cb1,
             vb0, vb1, rw0, rw1, acc, gs0, gs1):
        cid = lax.axis_index("c")
        sid = lax.axis_index("s")
        wbase = sid * cpw
        roff = cid * n_src

        def scale(rw, vb):
            for j0 in range(0, k, 16):
                vgrp = vb[pl.ds(j0, 16)]
                for jj in range(16):
                    bv = _bcast16(vgrp, jj)
                    for c0 in range(0, d, 16):
                        rw.at[j0 + jj][pl.ds(c0, 16)] = (
                            rw[j0 + jj, pl.ds(c0, 16)] * bv)

        def issue(c, rb, cb, vb, rw, gs):
            base = pl.multiple_of(c * k, k)
            pltpu.sync_copy(row_h.at[pl.ds(base, k)], rb)
            pltpu.sync_copy(col_h.at[pl.ds(base, k)], cb)
            pltpu.sync_copy(val_h.at[pl.ds(base, k)], vb)
            for j0 in range(0, k, 16):
                sl = pl.ds(j0, 16)
                rb[sl] = rb[sl] + roff
            pltpu.async_copy(h_h.at[rb], rw, gs)

        def consume(rb, cb, vb, rw, gs):
            pltpu.make_async_copy(h_h.at[rb], rw, gs).wait()
            scale(rw, vb)
            pltpu.sync_copy(rw, acc.at[cb], add=True)

        zz = jnp.zeros((16,), F32)
        for j in range(k):
            for c0 in range(0, d, 16):
                rw0.at[j][pl.ds(c0, 16)] = zz

        def zbody(i, cy):
            c = i * 16 + sid

            @pl.when(c < tot)
            def _():
                pltpu.sync_copy(rw0, acc.at[pl.ds(pl.multiple_of(c * k, k), k)])
            return cy

        lax.fori_loop(0, nloop, zbody, 0)
        if rem:
            @pl.when(sid == rem_tile)
            def _():
                pltpu.sync_copy(rw0.at[pl.ds(0, rem)],
                                acc.at[pl.ds(tot * k, rem)])
        plsc.subcore_barrier()

        if npairs:
            issue(wbase, rb0, cb0, vb0, rw0, gs0)
            issue(wbase + 1, rb1, cb1, vb1, rw1, gs1)

            def chunk2(i2, cy):
                c0 = wbase + i2 * 2
                consume(rb0, cb0, vb0, rw0, gs0)
                issue(c0 + 2, rb0, cb0, vb0, rw0, gs0)
                consume(rb1, cb1, vb1, rw1, gs1)
                issue(c0 + 3, rb1, cb1, vb1, rw1, gs1)
                return cy

            lax.fori_loop(0, npairs - 1, chunk2, 0)
            consume(rb0, cb0, vb0, rw0, gs0)
            consume(rb1, cb1, vb1, rw1, gs1)
        if odd:
            issue(wbase + cpw - 1, rb0, cb0, vb0, rw0, gs0)
            consume(rb0, cb0, vb0, rw0, gs0)
        plsc.subcore_barrier()

        def wbody(i, cy):
            c = i * 16 + sid

            @pl.when(c < tot)
            def _():
                r0 = pl.multiple_of(c * k, k)
                pltpu.sync_copy(acc.at[pl.ds(r0, k)], rw0)
                pltpu.sync_copy(
                    rw0, out_h.at[pl.ds(pl.multiple_of(cid * n_pad + r0, 8), k)])
            return cy

        lax.fori_loop(0, nloop, wbody, 0)
        if rem:
            @pl.when(sid == rem_tile)
            def _():
                pltpu.sync_copy(acc.at[pl.ds(tot * k, rem)],
                                rw0.at[pl.ds(0, rem)])
                pltpu.sync_copy(
                    rw0.at[pl.ds(0, rem)],
                    out_h.at[pl.ds(pl.multiple_of(cid * n_pad + tot * k, 8), rem)])

    return kern(hcat, row, col, val)


def _sc_deg(col, val, n_pad):
    e = col.shape[0]
    k = 128
    d = 16
    assert e % (k * 32) == 0
    cpw = e // (k * 32)
    tot, rem = divmod(n_pad, k)
    nloop = -(-tot // 16)
    rem_tile = tot % 16

    @functools.partial(
        pl.kernel,
        out_type=jax.ShapeDtypeStruct((2 * n_pad, d), F32),
        mesh=plsc.VectorSubcoreMesh(**_MESH),
        compiler_params=pltpu.CompilerParams(use_tc_tiling_on_sc=False),
        scratch_types=[
            pltpu.VMEM((k,), I32),
            pltpu.VMEM((k,), F32),
            pltpu.VMEM((k, d), F32),
            pltpu.VMEM_SHARED((n_pad, d), F32),
        ],
    )
    def kern(col_h, val_h, out_h, cbuf, vbuf, rows, acc):
        cid = lax.axis_index("c")
        sid = lax.axis_index("s")
        wid = cid * 16 + sid
        zz = jnp.zeros((16,), F32)
        for j in range(k):
            rows.at[j][pl.ds(0, 16)] = zz

        def zbody(i, cy):
            c = i * 16 + sid

            @pl.when(c < tot)
            def _():
                pltpu.sync_copy(rows, acc.at[pl.ds(pl.multiple_of(c * k, k), k)])
            return cy

        lax.fori_loop(0, nloop, zbody, 0)
        if rem:
            @pl.when(sid == rem_tile)
            def _():
                pltpu.sync_copy(rows.at[pl.ds(0, rem)],
                                acc.at[pl.ds(tot * k, rem)])
        plsc.subcore_barrier()

        def chunk(i, cy):
            base = pl.multiple_of((wid * cpw + i) * k, k)
            pltpu.sync_copy(col_h.at[pl.ds(base, k)], cbuf)
            pltpu.sync_copy(val_h.at[pl.ds(base, k)], vbuf)
            for j0 in range(0, k, 16):
                vgrp = vbuf[pl.ds(j0, 16)]
                for jj in range(16):
                    rows.at[j0 + jj][pl.ds(0, 16)] = _bcast16(vgrp, jj)
            pltpu.sync_copy(rows, acc.at[cbuf], add=True)
            return cy

        lax.fori_loop(0, cpw, chunk, 0)
        plsc.subcore_barrier()

        def wbody(i, cy):
            c = i * 16 + sid

            @pl.when(c < tot)
            def _():
                r0 = pl.multiple_of(c * k, k)
                pltpu.sync_copy(acc.at[pl.ds(r0, k)], rows)
                pltpu.sync_copy(
                    rows, out_h.at[pl.ds(pl.multiple_of(cid * n_pad + r0, 8), k)])
            return cy

        lax.fori_loop(0, nloop, wbody, 0)
        if rem:
            @pl.when(sid == rem_tile)
            def _():
                pltpu.sync_copy(acc.at[pl.ds(tot * k, rem)],
                                rows.at[pl.ds(0, rem)])
                pltpu.sync_copy(
                    rows.at[pl.ds(0, rem)],
                    out_h.at[pl.ds(pl.multiple_of(cid * n_pad + tot * k, 8), rem)])

    return kern(col, val)


def _sc_norm(row, col, w, dinv):
    e = row.shape[0]
    k = 128
    assert e % (k * 32) == 0
    cpw = e // (k * 32)

    @functools.partial(
        pl.kernel,
        out_type=jax.ShapeDtypeStruct((e,), F32),
        mesh=plsc.VectorSubcoreMesh(**_MESH),
        compiler_params=pltpu.CompilerParams(use_tc_tiling_on_sc=False),
        scratch_types=[
            pltpu.VMEM((k,), I32),
            pltpu.VMEM((k,), I32),
            pltpu.VMEM((k,), F32),
            pltpu.VMEM((k,), F32),
            pltpu.VMEM((k,), F32),
            pltpu.VMEM((k,), F32),
        ],
    )
    def kern(row_h, col_h, w_h, dinv_h, out_h, rbuf, cbuf, vbuf, nrb, ncb,
             obuf):
        cid = lax.axis_index("c")
        sid = lax.axis_index("s")
        wid = cid * 16 + sid

        def chunk(i, cy):
            base = pl.multiple_of((wid * cpw + i) * k, k)
            pltpu.sync_copy(row_h.at[pl.ds(base, k)], rbuf)
            pltpu.sync_copy(col_h.at[pl.ds(base, k)], cbuf)
            pltpu.sync_copy(w_h.at[pl.ds(base, k)], vbuf)
            pltpu.sync_copy(dinv_h.at[rbuf], nrb)
            pltpu.sync_copy(dinv_h.at[cbuf], ncb)
            for j0 in range(0, k, 16):
                sl = pl.ds(j0, 16)
                obuf[sl] = nrb[sl] * vbuf[sl] * ncb[sl]
            pltpu.sync_copy(obuf, out_h.at[pl.ds(base, k)])
            return cy

        lax.fori_loop(0, cpw, chunk, 0)

    return kern(row, col, w, dinv)


def _r2(a):
    return a.reshape(1, -1)


def _tc_fus(x, w, b, g, be, mw):
    n, din = x.shape
    dout = w.shape[1]
    blk = _bs(n)

    def body(x_r, w_r, b_r, g_r, be_r, mw_r, o_r):
        xx = jnp.nan_to_num(x_r[...])
        h = jnp.dot(xx, w_r[...], preferred_element_type=F32) + b_r[...]
        h = _gelu(_lnb(h, g_r[...], be_r[...]))
        nn = jnp.sqrt(jnp.sum(h * h, -1, keepdims=True))
        h = h / jnp.maximum(nn, 1e-8)
        m = mw_r[0, 0]
        h = h * jnp.exp(m - m)
        nn = jnp.sqrt(jnp.sum(h * h, -1, keepdims=True))
        o_r[...] = h / jnp.maximum(nn, 1e-8)

    return pl.pallas_call(
        body,
        grid=(n // blk,),
        in_specs=[
            pl.BlockSpec((blk, din), lambda i: (i, 0)),
            pl.BlockSpec((din, dout), lambda i: (0, 0)),
            pl.BlockSpec((1, dout), lambda i: (0, 0)),
            pl.BlockSpec((1, dout), lambda i: (0, 0)),
            pl.BlockSpec((1, dout), lambda i: (0, 0)),
            pl.BlockSpec((1, 1), lambda i: (0, 0)),
        ],
        out_specs=pl.BlockSpec((blk, dout), lambda i: (i, 0)),
        out_shape=jax.ShapeDtypeStruct((n, dout), F32),
    )(x, w, _r2(b), _r2(g), _r2(be), mw.reshape(1, 1))


def _tc_mm(x, w, skip=None, halves=False):
    n, din = x.shape
    dout = w.shape[1]
    blk = _bs(n)
    nblk = n // blk
    dh = dout // 2

    def body(*refs):
        if skip is not None:
            x_r, s_r, w_r = refs[0], refs[1], refs[2]
            o_r = refs[3]
            xx = x_r[...] + s_r[...]
        else:
            x_r, w_r = refs[0], refs[1]
            o_r = refs[2]
            xx = x_r[...]
        o_r[...] = jnp.dot(xx, w_r[...], preferred_element_type=F32)

    if halves:
        def hbody(*refs):
            if skip is not None:
                x_r, s_r, w_r, o_r = refs
                xx = x_r[...] + s_r[...]
            else:
                x_r, w_r, o_r = refs
                xx = x_r[...]
            full = jnp.dot(xx, w_r[...], preferred_element_type=F32)
            hsel = pl.program_id(0)
            o_r[...] = jnp.where(hsel == 0, full[:, :dh], full[:, dh:])

        in_specs = [pl.BlockSpec((blk, din), lambda h, i: (i, 0))]
        args = [x]
        if skip is not None:
            in_specs.append(pl.BlockSpec((blk, din), lambda h, i: (i, 0)))
            args.append(skip)
        in_specs.append(pl.BlockSpec((din, dout), lambda h, i: (0, 0)))
        args.append(w)
        return pl.pallas_call(
            hbody, grid=(2, nblk), in_specs=in_specs,
            out_specs=pl.BlockSpec((blk, dh),
                                   lambda h, i, nb_=nblk: (h * nb_ + i, 0)),
            out_shape=jax.ShapeDtypeStruct((2 * n, dh), F32))(*args)
    in_specs = [pl.BlockSpec((blk, din), lambda i: (i, 0))]
    args = [x]
    if skip is not None:
        in_specs.append(pl.BlockSpec((blk, din), lambda i: (i, 0)))
        args.append(skip)
    in_specs.append(pl.BlockSpec((din, dout), lambda i: (0, 0)))
    args.append(w)
    return pl.pallas_call(
        body, grid=(nblk,), in_specs=in_specs,
        out_specs=pl.BlockSpec((blk, dout), lambda i: (i, 0)),
        out_shape=jax.ShapeDtypeStruct((n, dout), F32))(*args)


def _tc_post(parts, hs, dinv2, bias, ng, nb, idn=None, split=False):
    n2, dh = parts[0].shape
    n = n2 // 2
    d = dh * (2 if split else 1) * len(parts)
    blk = _bs(n)
    nblk = n // blk
    np_ = len(parts)
    npr = 2 * np_
    nh = len(hs) * (2 if split else 1)

    def body(*refs):
        prefs = refs[:npr]
        hrefs = refs[npr:npr + nh]
        dv_r, b_r, g_r, nb_r = refs[npr + nh:npr + nh + 4]
        rest = refs[npr + nh + 4:]
        if split:
            pieces = [r[...] for r in prefs]
        else:
            pieces = [prefs[2 * t][...] + prefs[2 * t + 1][...]
                      for t in range(np_)]
        s = pieces[0] if len(pieces) == 1 else jnp.concatenate(pieces, axis=-1)
        hvals = [r[...] for r in hrefs]
        h = hvals[0] if nh == 1 else jnp.concatenate(hvals, axis=-1)
        dv = dv_r[...][:, 0:1]
        y = _gelu(_lnb(s + h * (dv * dv) + b_r[...], g_r[...], nb_r[...]))
        if idn is not None:
            y = y + rest[0][...]
        rest[-1][...] = y

    in_specs = []
    args = []
    for pt in parts:
        in_specs.append(pl.BlockSpec((blk, dh), lambda i: (i, 0)))
        args.append(pt)
        in_specs.append(
            pl.BlockSpec((blk, dh), lambda i, nb_=nblk: (i + nb_, 0)))
        args.append(pt)
    for hv in hs:
        in_specs.append(pl.BlockSpec((blk, dh), lambda i: (i, 0)))
        args.append(hv)
        if split:
            in_specs.append(
                pl.BlockSpec((blk, dh), lambda i, nb_=nblk: (i + nb_, 0)))
            args.append(hv)
    in_specs.append(pl.BlockSpec((blk, 16), lambda i: (i, 0)))
    args.append(dinv2)
    for v in (bias, ng, nb):
        in_specs.append(pl.BlockSpec((1, d), lambda i: (0, 0)))
        args.append(_r2(v))
    if idn is not None:
        in_specs.append(pl.BlockSpec((blk, d), lambda i: (i, 0)))
        args.append(idn)
    return pl.pallas_call(
        body, grid=(nblk,), in_specs=in_specs,
        out_specs=pl.BlockSpec((blk, d), lambda i: (i, 0)),
        out_shape=jax.ShapeDtypeStruct((n, d), F32),
    )(*args)


def _tc_mlp(xs, w, b, g, be, mode):
    if mode in ('plain', 'skip'):
        n, din = xs[0].shape
    elif mode == 'part':
        n = xs[0].shape[0] // 2
        din = xs[0].shape[1]
    elif mode == 'cat2':
        n = xs[0].shape[0] // 2
        din = xs[0].shape[1] * 2
    else:
        n = xs[0].shape[0] // 2
        din = xs[0].shape[1] * 2
    dout = w.shape[1]
    blk = _bs(n)
    nblk = n // blk

    def body(*refs):
        if mode == 'plain':
            xx = refs[0][...]
            k0 = 1
        elif mode == 'skip':
            xx = refs[0][...] + refs[1][...]
            k0 = 2
        elif mode == 'part':
            xx = refs[0][...] + refs[1][...]
            k0 = 2
        elif mode == 'cat2':
            xx = jnp.concatenate([refs[0][...], refs[1][...]], axis=-1)
            k0 = 2
        else:
            xx = jnp.concatenate(
                [refs[0][...] + refs[1][...], refs[2][...] + refs[3][...]],
                axis=-1)
            k0 = 4
        w_r, b_r, g_r, be_r, o_r = refs[k0:k0 + 5]
        h = jnp.dot(xx, w_r[...], preferred_element_type=F32) + b_r[...]
        o_r[...] = _gelu(_lnb(h, g_r[...], be_r[...]))

    in_specs = []
    args = []
    if mode == 'plain':
        in_specs.append(pl.BlockSpec((blk, din), lambda i: (i, 0)))
        args.append(xs[0])
    elif mode == 'skip':
        for a in xs:
            in_specs.append(pl.BlockSpec((blk, din), lambda i: (i, 0)))
            args.append(a)
    elif mode == 'part':
        in_specs.append(pl.BlockSpec((blk, din), lambda i: (i, 0)))
        args.append(xs[0])
        in_specs.append(pl.BlockSpec((blk, din), lambda i, nb_=nblk: (i + nb_, 0)))
        args.append(xs[0])
    elif mode == 'cat2':
        dh = din // 2
        in_specs.append(pl.BlockSpec((blk, dh), lambda i: (i, 0)))
        args.append(xs[0])
        in_specs.append(pl.BlockSpec((blk, dh), lambda i, nb_=nblk: (i + nb_, 0)))
        args.append(xs[0])
    else:
        dh = din // 2
        for a in xs:
            in_specs.append(pl.BlockSpec((blk, dh), lambda i: (i, 0)))
            args.append(a)
            in_specs.append(pl.BlockSpec((blk, dh), lambda i, nb_=nblk: (i + nb_, 0)))
            args.append(a)
    in_specs.append(pl.BlockSpec((din, dout), lambda i: (0, 0)))
    args.append(w)
    for v in (b, g, be):
        in_specs.append(pl.BlockSpec((1, dout), lambda i: (0, 0)))
        args.append(_r2(v))
    return pl.pallas_call(
        body, grid=(nblk,), in_specs=in_specs,
        out_specs=pl.BlockSpec((blk, dout), lambda i: (i, 0)),
        out_shape=jax.ShapeDtypeStruct((n, dout), F32),
    )(*args)


def _tc_proj(y, idn, w, b, g, be, halves=False):
    n, din = y.shape
    dout = w.shape[1]
    blk = _bs(n)
    dh = dout // 2

    def body(y_r, id_r, w_r, b_r, g_r, be_r, *outs):
        h = jnp.dot(y_r[...], w_r[...], preferred_element_type=F32) + b_r[...]
        out = _lnb(h, g_r[...], be_r[...]) + id_r[...]
        outs[0][...] = out
        if halves:
            outs[1][...] = out[:, :dh]
            outs[2][...] = out[:, dh:]

    out_specs = [pl.BlockSpec((blk, dout), lambda i: (i, 0))]
    out_shape = [jax.ShapeDtypeStruct((n, dout), F32)]
    if halves:
        out_specs += [pl.BlockSpec((blk, dh), lambda i: (i, 0))] * 2
        out_shape += [jax.ShapeDtypeStruct((n, dh), F32)] * 2
    res = pl.pallas_call(
        body, grid=(n // blk,),
        in_specs=[
            pl.BlockSpec((blk, din), lambda i: (i, 0)),
            pl.BlockSpec((blk, dout), lambda i: (i, 0)),
            pl.BlockSpec((din, dout), lambda i: (0, 0)),
            pl.BlockSpec((1, dout), lambda i: (0, 0)),
            pl.BlockSpec((1, dout), lambda i: (0, 0)),
            pl.BlockSpec((1, dout), lambda i: (0, 0)),
        ],
        out_specs=out_specs if halves else out_specs[0],
        out_shape=out_shape if halves else out_shape[0],
    )(y, idn, w, _r2(b), _r2(g), _r2(be))
    return res


def _tc_dinv(degp, n_pad):
    f = n_pad * 16
    x = degp.reshape(2, f)
    blk = 6400 if f % 6400 == 0 else 4096

    def body(p_r, o_r):
        xx = p_r[...]
        o_r[...] = lax.rsqrt(1.0 + xx[0:1, :] + xx[1:2, :])

    out = pl.pallas_call(
        body, grid=(f // blk,),
        in_specs=[pl.BlockSpec((2, blk), lambda i: (0, i))],
        out_specs=pl.BlockSpec((1, blk), lambda i: (0, i)),
        out_shape=jax.ShapeDtypeStruct((1, f), F32),
    )(x)
    return out.reshape(n_pad, 16)


def _tc_head(x, w, b):
    n, din = x.shape
    dout = w.shape[1]
    blk = _bs(n)

    def body(x_r, w_r, b_r, o_r):
        o_r[...] = jnp.dot(x_r[...], w_r[...],
                           preferred_element_type=F32) + b_r[...]

    return pl.pallas_call(
        body, grid=(n // blk,),
        in_specs=[
            pl.BlockSpec((blk, din), lambda i: (i, 0)),
            pl.BlockSpec((din, dout), lambda i: (0, 0)),
            pl.BlockSpec((1, dout), lambda i: (0, 0)),
        ],
        out_specs=pl.BlockSpec((blk, dout), lambda i: (i, 0)),
        out_shape=jax.ShapeDtypeStruct((n, dout), F32),
    )(x, w, _r2(b))


def _tc_head2(x, w1, b1, w2, b2):
    n, din = x.shape
    d1, d2 = w1.shape[1], w2.shape[1]
    blk = _bs(n)

    def body(x_r, w1_r, b1_r, w2_r, b2_r, o1_r, o2_r):
        xx = x_r[...]
        o1_r[...] = jnp.dot(xx, w1_r[...], preferred_element_type=F32) + b1_r[...]
        o2_r[...] = jnp.dot(xx, w2_r[...], preferred_element_type=F32) + b2_r[...]

    return pl.pallas_call(
        body, grid=(n // blk,),
        in_specs=[
            pl.BlockSpec((blk, din), lambda i: (i, 0)),
            pl.BlockSpec((din, d1), lambda i: (0, 0)),
            pl.BlockSpec((1, d1), lambda i: (0, 0)),
            pl.BlockSpec((din, d2), lambda i: (0, 0)),
            pl.BlockSpec((1, d2), lambda i: (0, 0)),
        ],
        out_specs=[
            pl.BlockSpec((blk, d1), lambda i: (i, 0)),
            pl.BlockSpec((blk, d2), lambda i: (i, 0)),
        ],
        out_shape=[
            jax.ShapeDtypeStruct((n, d1), F32),
            jax.ShapeDtypeStruct((n, d2), F32),
        ],
    )(x, w1, _r2(b1), w2, _r2(b2))


def _conv_block(x, skip, rowp, colp, normp, dinv2, p, pre, n_pad, d, fine,
                add_idn_last, idn):
    out = x
    for i in range(2):
        w = p[pre + '_convW' + str(i)]
        b = p[pre + '_convb' + str(i)]
        ng = p[pre + '_ng' + str(i)]
        nb = p[pre + '_nb' + str(i)]
        sk = skip if i == 0 else None
        last = add_idn_last and i == 1
        if fine:
            hcat = _tc_mm(out, w, skip=sk, halves=True)
            oflat = _sc_scatter_split(hcat, rowp, colp, normp, n_pad,
                                      d // 2, n_pad)
            out = _tc_post([oflat], [hcat], dinv2, b, ng, nb,
                           idn=idn if last else None, split=True)
        else:
            h = _tc_mm(out, w, skip=sk)
            pp = _sc_scatter(h, rowp, colp, normp, n_pad, d)
            out = _tc_post([pp], [h], dinv2, b, ng, nb,
                           idn=idn if last else None)
    return out


def _encoder(x, ed, p, pre, n_pad, d, fine):
    idn = _tc_mlp((x,), p[pre + '_res_W'], p[pre + '_res_b'],
                  p[pre + '_res_g'], p[pre + '_res_beta'], 'plain')
    return _conv_block(x, None, ed['row'], ed['col'], ed['norm'], ed['dinv2'],
                       p, pre, n_pad, d, fine, True, idn)


def _decoder(x, skip, ed, p, pre, n_pad, d, fine, proj, halves=False):
    idn = _tc_mlp((x, skip), p[pre + '_res_W'], p[pre + '_res_b'],
                  p[pre + '_res_g'], p[pre + '_res_beta'], 'skip')
    out = _conv_block(x, skip, ed['row'], ed['col'], ed['norm'], ed['dinv2'],
                      p, pre, n_pad, d, fine, not proj, idn)
    if proj:
        return _tc_proj(out, idn, p[pre + '_p_W'], p[pre + '_p_b'],
                        p[pre + '_p_g'], p[pre + '_p_beta'], halves=halves)
    return out


def _edge_level(ei, ew, n, n_pad):
    row = _pad1(ei[0], 4096)
    col = _pad1(ei[1], 4096)
    w = _pad1(ew, 4096)
    degp = _sc_deg(col, w, n_pad)
    dinv2 = _tc_dinv(degp, n_pad)
    dinv1 = dinv2[:, 0]
    norm = _sc_norm(row, col, w, dinv1)
    return dict(row=row, col=col, norm=norm, dinv2=dinv2, dinv1=dinv1)


def kernel(features, edge_index_fine, edge_weight_fine, edge_index_mid,
           edge_weight_mid, edge_index_coarse, edge_weight_coarse,
           map_f2m_idx, map_f2m_val, map_m2c_idx, map_m2c_val, map_c2m_idx,
           map_c2m_val, map_m2f_idx, map_m2f_val, params):
    p = params

    fused = _tc_fus(features, p['fus_W'], p['fus_b'], p['fus_g'],
                    p['fus_beta'], p['fus_mw'])

    edf = _edge_level(edge_index_fine, edge_weight_fine, NFINE, NFINE)
    edm = _edge_level(edge_index_mid, edge_weight_mid, NMID, NMIDP)
    edc = _edge_level(edge_index_coarse, edge_weight_coarse, NCOARSE, NCOARSEP)

    e1 = _encoder(fused, edf, p, 'enc1', NFINE, 64, True)

    f2m_r = _pad1(map_f2m_idx[1], 4096)
    f2m_c = _pad1(map_f2m_idx[0], 4096)
    f2m_v = _pad1(map_f2m_val, 4096)
    m1p = _sc_scatter(e1, f2m_r, f2m_c, f2m_v, NMIDP, 64)
    m1 = _tc_mlp((m1p,), p['f2m_W'], p['f2m_b'], p['f2m_g'], p['f2m_beta'],
                 'part')

    e2 = _encoder(m1, edm, p, 'enc2', NMIDP, 128, False)

    m2c_r = _pad1(map_m2c_idx[1], 4096)
    m2c_c = _pad1(map_m2c_idx[0], 4096)
    m2c_v = _pad1(map_m2c_val, 4096)
    m2p = _sc_scatter(e2, m2c_r, m2c_c, m2c_v, NCOARSEP, 128)
    m2 = _tc_mlp((m2p,), p['m2c_W'], p['m2c_b'], p['m2c_g'], p['m2c_beta'],
                 'part')

    e3 = _encoder(m2, edc, p, 'enc3', NCOARSEP, 256, False)

    d3 = _decoder(e3, m2, edc, p, 'dec3', NCOARSEP, 256, False, True)

    c2m_r = _pad1(map_c2m_idx[1], 4096)
    c2m_c = _pad1(map_c2m_idx[0], 4096)
    c2m_v = _pad1(map_c2m_val, 4096)
    u2p = _sc_scatter(d3, c2m_r, c2m_c, c2m_v, NMIDP, 128)
    u2 = _tc_mlp((u2p,), p['c2m_W'], p['c2m_b'], p['c2m_g'], p['c2m_beta'],
                 'part')

    d2, d2a, d2b = _decoder(u2, e2, edm, p, 'dec2', NMIDP, 128, False, True,
                            halves=True)

    m2f_r = _pad1(map_m2f_idx[1], 4096)
    m2f_c = _pad1(map_m2f_idx[0], 4096)
    m2f_v = _pad1(map_m2f_val, 4096)
    d2h = jnp.concatenate([d2a, d2b], axis=0)
    u1flat = _sc_scatter_split(d2h, m2f_r, m2f_c, m2f_v, NFINE, 32, NMIDP)
    u1 = _tc_mlp((u1flat,), p['m2f_W'], p['m2f_b'], p['m2f_g'],
                 p['m2f_beta'], 'cat2')

    d1 = _decoder(u1, e1, edf, p, 'dec1', NFINE, 64, True, False)

    emb_f, recon = _tc_head2(d1, p['head_f_W'], p['head_f_b'], p['rec_W'],
                             p['rec_b'])
    emb_m = _tc_head(d2, p['head_m_W'], p['head_m_b'])[:NMID]
    emb_c = _tc_head(d3, p['head_c_W'], p['head_c_b'])[:NCOARSE]
    return emb_f, emb_m, emb_c, recon


# speedup vs baseline: 7.5571x; 1.0597x over previous
import functools

import jax
import jax.numpy as jnp
from jax import lax
from jax.experimental import pallas as pl
from jax.experimental.pallas import tpu as pltpu
from jax.experimental.pallas import tpu_sc as plsc

F32 = jnp.float32
I32 = jnp.int32

NFINE, NMID, NCOARSE = 50000, 7143, 1020
NMIDP, NCOARSEP = 7168, 1024
DIN = 208

_MESH = dict(core_axis_name="c", subcore_axis_name="s", num_cores=2,
             num_subcores=16)


def _bs(n):
    return 1000 if n == NFINE else 1024


_BCAST_DNUMS = lax.GatherDimensionNumbers(
    offset_dims=(), collapsed_slice_dims=(0,), start_index_map=(0,))


def _bcast16(vec, j):
    return lax.gather(vec, jnp.full((16, 1), j, I32), _BCAST_DNUMS, (1,),
                      mode=lax.GatherScatterMode.PROMISE_IN_BOUNDS)


def _gelu(x):
    return 0.5 * x * (1.0 + lax.erf(x * 0.7071067811865476))


def _lnb(x, g, b):
    m = jnp.mean(x, axis=-1, keepdims=True)
    v = jnp.mean((x - m) ** 2, axis=-1, keepdims=True)
    return (x - m) / jnp.sqrt(v + 1e-5) * g + b


def _pad1(a, m):
    e = a.shape[0]
    ep = -(-e // m) * m
    if ep == e:
        return a
    return jnp.pad(a, ((0, ep - e),))


def _k_for_d(d):
    if d <= 64:
        return 128
    if d == 128:
        return 64
    return 32


def _sc_scatter(h, row, col, val, n_pad, d):
    e = row.shape[0]
    k = _k_for_d(d)
    assert e % (k * 32) == 0, (e, k)
    cpw = e // (k * 32)
    npairs = cpw // 2
    odd = cpw % 2
    tot, rem = divmod(n_pad, k)
    nloop = -(-tot // 16)
    rem_tile = tot % 16

    @functools.partial(
        pl.kernel,
        out_type=jax.ShapeDtypeStruct((2 * n_pad, d), F32),
        mesh=plsc.VectorSubcoreMesh(**_MESH),
        compiler_params=pltpu.CompilerParams(use_tc_tiling_on_sc=False),
        scratch_types=[
            pltpu.VMEM((k,), I32),
            pltpu.VMEM((k,), I32),
            pltpu.VMEM((k,), I32),
            pltpu.VMEM((k,), I32),
            pltpu.VMEM((k,), F32),
            pltpu.VMEM((k,), F32),
            pltpu.VMEM((k, d), F32),
            pltpu.VMEM((k, d), F32),
            pltpu.VMEM_SHARED((n_pad, d), F32),
            pltpu.SemaphoreType.DMA,
            pltpu.SemaphoreType.DMA,
        ],
    )
    def kern(h_h, row_h, col_h, val_h, out_h, rb0, rb1, cb0, cb1, vb0, vb1,
             rw0, rw1, acc, gs0, gs1):
        cid = lax.axis_index("c")
        sid = lax.axis_index("s")
        wid = cid * 16 + sid
        wbase = wid * cpw

        def scale(rw, vb):
            for j0 in range(0, k, 16):
                vgrp = vb[pl.ds(j0, 16)]
                for jj in range(16):
                    bv = _bcast16(vgrp, jj)
                    for c0 in range(0, d, 16):
                        rw.at[j0 + jj][pl.ds(c0, 16)] = (
                            rw[j0 + jj, pl.ds(c0, 16)] * bv)

        def issue(c, rb, cb, vb, rw, gs):
            base = pl.multiple_of(c * k, k)
            pltpu.sync_copy(row_h.at[pl.ds(base, k)], rb)
            pltpu.sync_copy(col_h.at[pl.ds(base, k)], cb)
            pltpu.sync_copy(val_h.at[pl.ds(base, k)], vb)
            pltpu.async_copy(h_h.at[rb], rw, gs)

        def consume(rb, cb, vb, rw, gs):
            pltpu.make_async_copy(h_h.at[rb], rw, gs).wait()
            scale(rw, vb)
            pltpu.sync_copy(rw, acc.at[cb], add=True)

        zz = jnp.zeros((16,), F32)
        for j in range(k):
            for c0 in range(0, d, 16):
                rw0.at[j][pl.ds(c0, 16)] = zz

        def zbody(i, cy):
            c = i * 16 + sid

            @pl.when(c < tot)
            def _():
                pltpu.sync_copy(rw0, acc.at[pl.ds(pl.multiple_of(c * k, k), k)])
            return cy

        lax.fori_loop(0, nloop, zbody, 0)
        if rem:
            @pl.when(sid == rem_tile)
            def _():
                pltpu.sync_copy(rw0.at[pl.ds(0, rem)],
                                acc.at[pl.ds(tot * k, rem)])
        plsc.subcore_barrier()

        if npairs:
            issue(wbase, rb0, cb0, vb0, rw0, gs0)
            issue(wbase + 1, rb1, cb1, vb1, rw1, gs1)

            def chunk2(i2, cy):
                c0 = wbase + i2 * 2
                consume(rb0, cb0, vb0, rw0, gs0)
                issue(c0 + 2, rb0, cb0, vb0, rw0, gs0)
                consume(rb1, cb1, vb1, rw1, gs1)
                issue(c0 + 3, rb1, cb1, vb1, rw1, gs1)
                return cy

            lax.fori_loop(0, npairs - 1, chunk2, 0)
            consume(rb0, cb0, vb0, rw0, gs0)
            consume(rb1, cb1, vb1, rw1, gs1)
        if odd:
            issue(wbase + cpw - 1, rb0, cb0, vb0, rw0, gs0)
            consume(rb0, cb0, vb0, rw0, gs0)
        plsc.subcore_barrier()

        def wbody(i, cy):
            c = i * 16 + sid

            @pl.when(c < tot)
            def _():
                r0 = pl.multiple_of(c * k, k)
                pltpu.sync_copy(acc.at[pl.ds(r0, k)], rw0)
                pltpu.sync_copy(
                    rw0, out_h.at[pl.ds(pl.multiple_of(cid * n_pad + r0, 8), k)])
            return cy

        lax.fori_loop(0, nloop, wbody, 0)
        if rem:
            @pl.when(sid == rem_tile)
            def _():
                pltpu.sync_copy(acc.at[pl.ds(tot * k, rem)],
                                rw0.at[pl.ds(0, rem)])
                pltpu.sync_copy(
                    rw0.at[pl.ds(0, rem)],
                    out_h.at[pl.ds(pl.multiple_of(cid * n_pad + tot * k, 8), rem)])

    return kern(h, row, col, val)


def _sc_scatter_split(hcat, row, col, val, n_pad, dh, n_src):
    e = row.shape[0]
    d = dh
    k = _k_for_d(d)
    assert e % (k * 16) == 0, (e, k)
    cpw = e // (k * 16)
    npairs = cpw // 2
    odd = cpw % 2
    tot, rem = divmod(n_pad, k)
    nloop = -(-tot // 16)
    rem_tile = tot % 16

    @functools.partial(
        pl.kernel,
        out_type=jax.ShapeDtypeStruct((2 * n_pad, dh), F32),
        mesh=plsc.VectorSubcoreMesh(**_MESH),
        compiler_params=pltpu.CompilerParams(use_tc_tiling_on_sc=False),
        scratch_types=[
            pltpu.VMEM((k,), I32),
            pltpu.VMEM((k,), I32),
            pltpu.VMEM((k,), I32),
            pltpu.VMEM((k,), I32),
            pltpu.VMEM((k,), F32),
            pltpu.VMEM((k,), F32),
            pltpu.VMEM((k, d), F32),
            pltpu.VMEM((k, d), F32),
            pltpu.VMEM_SHARED((n_pad, d), F32),
            pltpu.SemaphoreType.DMA,
            pltpu.SemaphoreType.DMA,
        ],
    )
    def kern(h_h, row_h, col_h, val_h, out_h, rb0, rb1, cb0, cb1,
             vb0, vb1, rw0, rw1, acc, gs0, gs1):
        cid = lax.axis_index("c")
        sid = lax.axis_index("s")
        wbase = sid * cpw
        roff = cid * n_src

        def scale(rw, vb):
            for j0 in range(0, k, 16):
                vgrp = vb[pl.ds(j0, 16)]
                for jj in range(16):
                    bv = _bcast16(vgrp, jj)
                    for c0 in range(0, d, 16):
                        rw.at[j0 + jj][pl.ds(c0, 16)] = (
                            rw[j0 + jj, pl.ds(c0, 16)] * bv)

        def issue(c, rb, cb, vb, rw, gs):
            base = pl.multiple_of(c * k, k)
            pltpu.sync_copy(row_h.at[pl.ds(base, k)], rb)
            pltpu.sync_copy(col_h.at[pl.ds(base, k)], cb)
            pltpu.sync_copy(val_h.at[pl.ds(base, k)], vb)
            for j0 in range(0, k, 16):
                sl = pl.ds(j0, 16)
                rb[sl] = rb[sl] + roff
            pltpu.async_copy(h_h.at[rb], rw, gs)

        def consume(rb, cb, vb, rw, gs):
            pltpu.make_async_copy(h_h.at[rb], rw, gs).wait()
            scale(rw, vb)
            pltpu.sync_copy(rw, acc.at[cb], add=True)

        zz = jnp.zeros((16,), F32)
        for j in range(k):
            for c0 in range(0, d, 16):
                rw0.at[j][pl.ds(c0, 16)] = zz

        def zbody(i, cy):
            c = i * 16 + sid

            @pl.when(c < tot)
            def _():
                pltpu.sync_copy(rw0, acc.at[pl.ds(pl.multiple_of(c * k, k), k)])
            return cy

        lax.fori_loop(0, nloop, zbody, 0)
        if rem:
            @pl.when(sid == rem_tile)
            def _():
                pltpu.sync_copy(rw0.at[pl.ds(0, rem)],
                                acc.at[pl.ds(tot * k, rem)])
        plsc.subcore_barrier()

        if npairs:
            issue(wbase, rb0, cb0, vb0, rw0, gs0)
            issue(wbase + 1, rb1, cb1, vb1, rw1, gs1)

            def chunk2(i2, cy):
                c0 = wbase + i2 * 2
                consume(rb0, cb0, vb0, rw0, gs0)
                issue(c0 + 2, rb0, cb0, vb0, rw0, gs0)
                consume(rb1, cb1, vb1, rw1, gs1)
                issue(c0 + 3, rb1, cb1, vb1, rw1, gs1)
                return cy

            lax.fori_loop(0, npairs - 1, chunk2, 0)
            consume(rb0, cb0, vb0, rw0, gs0)
            consume(rb1, cb1, vb1, rw1, gs1)
        if odd:
            issue(wbase + cpw - 1, rb0, cb0, vb0, rw0, gs0)
            consume(rb0, cb0, vb0, rw0, gs0)
        plsc.subcore_barrier()

        def wbody(i, cy):
            c = i * 16 + sid

            @pl.when(c < tot)
            def _():
                r0 = pl.multiple_of(c * k, k)
                pltpu.sync_copy(acc.at[pl.ds(r0, k)], rw0)
                pltpu.sync_copy(
                    rw0, out_h.at[pl.ds(pl.multiple_of(cid * n_pad + r0, 8), k)])
            return cy

        lax.fori_loop(0, nloop, wbody, 0)
        if rem:
            @pl.when(sid == rem_tile)
            def _():
                pltpu.sync_copy(acc.at[pl.ds(tot * k, rem)],
                                rw0.at[pl.ds(0, rem)])
                pltpu.sync_copy(
                    rw0.at[pl.ds(0, rem)],
                    out_h.at[pl.ds(pl.multiple_of(cid * n_pad + tot * k, 8), rem)])

    return kern(hcat, row, col, val)


def _sc_deg(col, val, n_pad):
    e = col.shape[0]
    k = 128
    d = 16
    assert e % (k * 32) == 0
    cpw = e // (k * 32)
    tot, rem = divmod(n_pad, k)
    nloop = -(-tot // 16)
    rem_tile = tot % 16

    @functools.partial(
        pl.kernel,
        out_type=jax.ShapeDtypeStruct((2 * n_pad, d), F32),
        mesh=plsc.VectorSubcoreMesh(**_MESH),
        compiler_params=pltpu.CompilerParams(use_tc_tiling_on_sc=False),
        scratch_types=[
            pltpu.VMEM((k,), I32),
            pltpu.VMEM((k,), I32),
            pltpu.VMEM((k,), F32),
            pltpu.VMEM((k,), F32),
            pltpu.VMEM((k, d), F32),
            pltpu.VMEM((k, d), F32),
            pltpu.VMEM_SHARED((n_pad, d), F32),
            pltpu.SemaphoreType.DMA,
            pltpu.SemaphoreType.DMA,
        ],
    )
    def kern(col_h, val_h, out_h, cbuf, cbuf1, vbuf, vbuf1, rows, rows1, acc,
             ss0, ss1):
        cid = lax.axis_index("c")
        sid = lax.axis_index("s")
        wid = cid * 16 + sid
        zz = jnp.zeros((16,), F32)
        for j in range(k):
            rows.at[j][pl.ds(0, 16)] = zz

        def zbody(i, cy):
            c = i * 16 + sid

            @pl.when(c < tot)
            def _():
                pltpu.sync_copy(rows, acc.at[pl.ds(pl.multiple_of(c * k, k), k)])
            return cy

        lax.fori_loop(0, nloop, zbody, 0)
        if rem:
            @pl.when(sid == rem_tile)
            def _():
                pltpu.sync_copy(rows.at[pl.ds(0, rem)],
                                acc.at[pl.ds(tot * k, rem)])
        plsc.subcore_barrier()

        def lin_bcast(c, cb, vb, rw):
            base = pl.multiple_of(c * k, k)
            pltpu.sync_copy(col_h.at[pl.ds(base, k)], cb)
            pltpu.sync_copy(val_h.at[pl.ds(base, k)], vb)
            for j0 in range(0, k, 16):
                vgrp = vb[pl.ds(j0, 16)]
                for jj in range(16):
                    rw.at[j0 + jj][pl.ds(0, 16)] = _bcast16(vgrp, jj)

        assert cpw % 2 == 0
        npairs = cpw // 2
        wbase = wid * cpw
        lin_bcast(wbase, cbuf, vbuf, rows)
        pltpu.async_copy(rows, acc.at[cbuf], ss0, add=True)
        lin_bcast(wbase + 1, cbuf1, vbuf1, rows1)
        pltpu.async_copy(rows1, acc.at[cbuf1], ss1, add=True)

        def chunk2(i2, cy):
            c0 = wbase + 2 + i2 * 2
            pltpu.make_async_copy(rows, acc.at[cbuf], ss0).wait()
            lin_bcast(c0, cbuf, vbuf, rows)
            pltpu.async_copy(rows, acc.at[cbuf], ss0, add=True)
            pltpu.make_async_copy(rows1, acc.at[cbuf1], ss1).wait()
            lin_bcast(c0 + 1, cbuf1, vbuf1, rows1)
            pltpu.async_copy(rows1, acc.at[cbuf1], ss1, add=True)
            return cy

        lax.fori_loop(0, npairs - 1, chunk2, 0)
        pltpu.make_async_copy(rows, acc.at[cbuf], ss0).wait()
        pltpu.make_async_copy(rows1, acc.at[cbuf1], ss1).wait()
        plsc.subcore_barrier()

        def wbody(i, cy):
            c = i * 16 + sid

            @pl.when(c < tot)
            def _():
                r0 = pl.multiple_of(c * k, k)
                pltpu.sync_copy(acc.at[pl.ds(r0, k)], rows)
                pltpu.sync_copy(
                    rows, out_h.at[pl.ds(pl.multiple_of(cid * n_pad + r0, 8), k)])
            return cy

        lax.fori_loop(0, nloop, wbody, 0)
        if rem:
            @pl.when(sid == rem_tile)
            def _():
                pltpu.sync_copy(acc.at[pl.ds(tot * k, rem)],
                                rows.at[pl.ds(0, rem)])
                pltpu.sync_copy(
                    rows.at[pl.ds(0, rem)],
                    out_h.at[pl.ds(pl.multiple_of(cid * n_pad + tot * k, 8), rem)])

    return kern(col, val)


def _sc_norm(row, col, w, dinv):
    e = row.shape[0]
    k = 128
    assert e % (k * 32) == 0
    cpw = e // (k * 32)

    npairs = cpw // 2
    odd = cpw % 2

    @functools.partial(
        pl.kernel,
        out_type=jax.ShapeDtypeStruct((e,), F32),
        mesh=plsc.VectorSubcoreMesh(**_MESH),
        compiler_params=pltpu.CompilerParams(use_tc_tiling_on_sc=False),
        scratch_types=[
            pltpu.VMEM((k,), I32),
            pltpu.VMEM((k,), I32),
            pltpu.VMEM((k,), I32),
            pltpu.VMEM((k,), I32),
            pltpu.VMEM((k,), F32),
            pltpu.VMEM((k,), F32),
            pltpu.VMEM((k,), F32),
            pltpu.VMEM((k,), F32),
            pltpu.VMEM((k,), F32),
            pltpu.VMEM((k,), F32),
            pltpu.VMEM((k,), F32),
            pltpu.VMEM((k,), F32),
            pltpu.SemaphoreType.DMA,
            pltpu.SemaphoreType.DMA,
        ],
    )
    def kern(row_h, col_h, w_h, dinv_h, out_h, rb0, rb1, cb0, cb1, vb0, vb1,
             nr0, nr1, nc0, nc1, ob0, ob1, g0, g1):
        cid = lax.axis_index("c")
        sid = lax.axis_index("s")
        wid = cid * 16 + sid
        wbase = wid * cpw

        def issue(c, rb, cb, vb, nr, nc, gs):
            base = pl.multiple_of(c * k, k)
            pltpu.sync_copy(row_h.at[pl.ds(base, k)], rb)
            pltpu.sync_copy(col_h.at[pl.ds(base, k)], cb)
            pltpu.sync_copy(w_h.at[pl.ds(base, k)], vb)
            pltpu.async_copy(dinv_h.at[rb], nr, gs)
            pltpu.async_copy(dinv_h.at[cb], nc, gs)

        def consume(c, rb, cb, vb, nr, nc, ob, gs):
            base = pl.multiple_of(c * k, k)
            pltpu.make_async_copy(dinv_h.at[rb], nr, gs).wait()
            pltpu.make_async_copy(dinv_h.at[cb], nc, gs).wait()
            for j0 in range(0, k, 16):
                sl = pl.ds(j0, 16)
                ob[sl] = nr[sl] * vb[sl] * nc[sl]
            pltpu.sync_copy(ob, out_h.at[pl.ds(base, k)])

        if npairs:
            issue(wbase, rb0, cb0, vb0, nr0, nc0, g0)
            issue(wbase + 1, rb1, cb1, vb1, nr1, nc1, g1)

            def chunk2(i2, cy):
                c0 = wbase + i2 * 2
                consume(c0, rb0, cb0, vb0, nr0, nc0, ob0, g0)
                issue(c0 + 2, rb0, cb0, vb0, nr0, nc0, g0)
                consume(c0 + 1, rb1, cb1, vb1, nr1, nc1, ob1, g1)
                issue(c0 + 3, rb1, cb1, vb1, nr1, nc1, g1)
                return cy

            lax.fori_loop(0, npairs - 1, chunk2, 0)
            consume(wbase + 2 * npairs - 2, rb0, cb0, vb0, nr0, nc0, ob0, g0)
            consume(wbase + 2 * npairs - 1, rb1, cb1, vb1, nr1, nc1, ob1, g1)
        if odd:
            issue(wbase + cpw - 1, rb0, cb0, vb0, nr0, nc0, g0)
            consume(wbase + cpw - 1, rb0, cb0, vb0, nr0, nc0, ob0, g0)

    return kern(row, col, w, dinv)


def _r2(a):
    return a.reshape(1, -1)


def _tc_fus(x, w, b, g, be, mw):
    n, din = x.shape
    dout = w.shape[1]
    blk = _bs(n)

    def body(x_r, w_r, b_r, g_r, be_r, mw_r, o_r):
        xx = jnp.nan_to_num(x_r[...])
        h = jnp.dot(xx, w_r[...], preferred_element_type=F32) + b_r[...]
        h = _gelu(_lnb(h, g_r[...], be_r[...]))
        nn = jnp.sqrt(jnp.sum(h * h, -1, keepdims=True))
        h = h / jnp.maximum(nn, 1e-8)
        m = mw_r[0, 0]
        h = h * jnp.exp(m - m)
        nn = jnp.sqrt(jnp.sum(h * h, -1, keepdims=True))
        o_r[...] = h / jnp.maximum(nn, 1e-8)

    return pl.pallas_call(
        body,
        grid=(n // blk,),
        in_specs=[
            pl.BlockSpec((blk, din), lambda i: (i, 0)),
            pl.BlockSpec((din, dout), lambda i: (0, 0)),
            pl.BlockSpec((1, dout), lambda i: (0, 0)),
            pl.BlockSpec((1, dout), lambda i: (0, 0)),
            pl.BlockSpec((1, dout), lambda i: (0, 0)),
            pl.BlockSpec((1, 1), lambda i: (0, 0)),
        ],
        out_specs=pl.BlockSpec((blk, dout), lambda i: (i, 0)),
        out_shape=jax.ShapeDtypeStruct((n, dout), F32),
    )(x, w, _r2(b), _r2(g), _r2(be), mw.reshape(1, 1))


def _tc_mm(x, w, skip=None, halves=False):
    n, din = x.shape
    dout = w.shape[1]
    blk = _bs(n)
    nblk = n // blk
    dh = dout // 2

    def body(*refs):
        if skip is not None:
            x_r, s_r, w_r = refs[0], refs[1], refs[2]
            o_r = refs[3]
            xx = x_r[...] + s_r[...]
        else:
            x_r, w_r = refs[0], refs[1]
            o_r = refs[2]
            xx = x_r[...]
        o_r[...] = jnp.dot(xx, w_r[...], preferred_element_type=F32)

    if halves:
        def hbody(*refs):
            if skip is not None:
                x_r, s_r, w_r, o_r = refs
                xx = x_r[...] + s_r[...]
            else:
                x_r, w_r, o_r = refs
                xx = x_r[...]
            full = jnp.dot(xx, w_r[...], preferred_element_type=F32)
            hsel = pl.program_id(0)
            o_r[...] = jnp.where(hsel == 0, full[:, :dh], full[:, dh:])

        in_specs = [pl.BlockSpec((blk, din), lambda h, i: (i, 0))]
        args = [x]
        if skip is not None:
            in_specs.append(pl.BlockSpec((blk, din), lambda h, i: (i, 0)))
            args.append(skip)
        in_specs.append(pl.BlockSpec((din, dout), lambda h, i: (0, 0)))
        args.append(w)
        return pl.pallas_call(
            hbody, grid=(2, nblk), in_specs=in_specs,
            out_specs=pl.BlockSpec((blk, dh),
                                   lambda h, i, nb_=nblk: (h * nb_ + i, 0)),
            out_shape=jax.ShapeDtypeStruct((2 * n, dh), F32))(*args)
    in_specs = [pl.BlockSpec((blk, din), lambda i: (i, 0))]
    args = [x]
    if skip is not None:
        in_specs.append(pl.BlockSpec((blk, din), lambda i: (i, 0)))
        args.append(skip)
    in_specs.append(pl.BlockSpec((din, dout), lambda i: (0, 0)))
    args.append(w)
    return pl.pallas_call(
        body, grid=(nblk,), in_specs=in_specs,
        out_specs=pl.BlockSpec((blk, dout), lambda i: (i, 0)),
        out_shape=jax.ShapeDtypeStruct((n, dout), F32))(*args)


def _tc_post(parts, hs, dinv2, bias, ng, nb, idn=None, split=False):
    n2, dh = parts[0].shape
    n = n2 // 2
    d = dh * (2 if split else 1) * len(parts)
    blk = _bs(n)
    nblk = n // blk
    np_ = len(parts)
    npr = 2 * np_
    nh = len(hs) * (2 if split else 1)

    def body(*refs):
        prefs = refs[:npr]
        hrefs = refs[npr:npr + nh]
        dv_r, b_r, g_r, nb_r = refs[npr + nh:npr + nh + 4]
        rest = refs[npr + nh + 4:]
        if split:
            pieces = [r[...] for r in prefs]
        else:
            pieces = [prefs[2 * t][...] + prefs[2 * t + 1][...]
                      for t in range(np_)]
        s = pieces[0] if len(pieces) == 1 else jnp.concatenate(pieces, axis=-1)
        hvals = [r[...] for r in hrefs]
        h = hvals[0] if nh == 1 else jnp.concatenate(hvals, axis=-1)
        dv = dv_r[...][:, 0:1]
        y = _gelu(_lnb(s + h * (dv * dv) + b_r[...], g_r[...], nb_r[...]))
        if idn is not None:
            y = y + rest[0][...]
        rest[-1][...] = y

    in_specs = []
    args = []
    for pt in parts:
        in_specs.append(pl.BlockSpec((blk, dh), lambda i: (i, 0)))
        args.append(pt)
        in_specs.append(
            pl.BlockSpec((blk, dh), lambda i, nb_=nblk: (i + nb_, 0)))
        args.append(pt)
    for hv in hs:
        in_specs.append(pl.BlockSpec((blk, dh), lambda i: (i, 0)))
        args.append(hv)
        if split:
            in_specs.append(
                pl.BlockSpec((blk, dh), lambda i, nb_=nblk: (i + nb_, 0)))
            args.append(hv)
    in_specs.append(pl.BlockSpec((blk, 16), lambda i: (i, 0)))
    args.append(dinv2)
    for v in (bias, ng, nb):
        in_specs.append(pl.BlockSpec((1, d), lambda i: (0, 0)))
        args.append(_r2(v))
    if idn is not None:
        in_specs.append(pl.BlockSpec((blk, d), lambda i: (i, 0)))
        args.append(idn)
    return pl.pallas_call(
        body, grid=(nblk,), in_specs=in_specs,
        out_specs=pl.BlockSpec((blk, d), lambda i: (i, 0)),
        out_shape=jax.ShapeDtypeStruct((n, d), F32),
    )(*args)


def _tc_mlp(xs, w, b, g, be, mode):
    if mode in ('plain', 'skip'):
        n, din = xs[0].shape
    elif mode == 'part':
        n = xs[0].shape[0] // 2
        din = xs[0].shape[1]
    elif mode == 'cat2':
        n = xs[0].shape[0] // 2
        din = xs[0].shape[1] * 2
    else:
        n = xs[0].shape[0] // 2
        din = xs[0].shape[1] * 2
    dout = w.shape[1]
    blk = _bs(n)
    nblk = n // blk

    def body(*refs):
        if mode == 'plain':
            xx = refs[0][...]
            k0 = 1
        elif mode == 'skip':
            xx = refs[0][...] + refs[1][...]
            k0 = 2
        elif mode == 'part':
            xx = refs[0][...] + refs[1][...]
            k0 = 2
        elif mode == 'cat2':
            xx = jnp.concatenate([refs[0][...], refs[1][...]], axis=-1)
            k0 = 2
        else:
            xx = jnp.concatenate(
                [refs[0][...] + refs[1][...], refs[2][...] + refs[3][...]],
                axis=-1)
            k0 = 4
        w_r, b_r, g_r, be_r, o_r = refs[k0:k0 + 5]
        h = jnp.dot(xx, w_r[...], preferred_element_type=F32) + b_r[...]
        o_r[...] = _gelu(_lnb(h, g_r[...], be_r[...]))

    in_specs = []
    args = []
    if mode == 'plain':
        in_specs.append(pl.BlockSpec((blk, din), lambda i: (i, 0)))
        args.append(xs[0])
    elif mode == 'skip':
        for a in xs:
            in_specs.append(pl.BlockSpec((blk, din), lambda i: (i, 0)))
            args.append(a)
    elif mode == 'part':
        in_specs.append(pl.BlockSpec((blk, din), lambda i: (i, 0)))
        args.append(xs[0])
        in_specs.append(pl.BlockSpec((blk, din), lambda i, nb_=nblk: (i + nb_, 0)))
        args.append(xs[0])
    elif mode == 'cat2':
        dh = din // 2
        in_specs.append(pl.BlockSpec((blk, dh), lambda i: (i, 0)))
        args.append(xs[0])
        in_specs.append(pl.BlockSpec((blk, dh), lambda i, nb_=nblk: (i + nb_, 0)))
        args.append(xs[0])
    else:
        dh = din // 2
        for a in xs:
            in_specs.append(pl.BlockSpec((blk, dh), lambda i: (i, 0)))
            args.append(a)
            in_specs.append(pl.BlockSpec((blk, dh), lambda i, nb_=nblk: (i + nb_, 0)))
            args.append(a)
    in_specs.append(pl.BlockSpec((din, dout), lambda i: (0, 0)))
    args.append(w)
    for v in (b, g, be):
        in_specs.append(pl.BlockSpec((1, dout), lambda i: (0, 0)))
        args.append(_r2(v))
    return pl.pallas_call(
        body, grid=(nblk,), in_specs=in_specs,
        out_specs=pl.BlockSpec((blk, dout), lambda i: (i, 0)),
        out_shape=jax.ShapeDtypeStruct((n, dout), F32),
    )(*args)


def _tc_proj(y, idn, w, b, g, be, halves=False):
    n, din = y.shape
    dout = w.shape[1]
    blk = _bs(n)
    dh = dout // 2

    def body(y_r, id_r, w_r, b_r, g_r, be_r, *outs):
        h = jnp.dot(y_r[...], w_r[...], preferred_element_type=F32) + b_r[...]
        out = _lnb(h, g_r[...], be_r[...]) + id_r[...]
        outs[0][...] = out
        if halves:
            outs[1][...] = out[:, :dh]
            outs[2][...] = out[:, dh:]

    out_specs = [pl.BlockSpec((blk, dout), lambda i: (i, 0))]
    out_shape = [jax.ShapeDtypeStruct((n, dout), F32)]
    if halves:
        out_specs += [pl.BlockSpec((blk, dh), lambda i: (i, 0))] * 2
        out_shape += [jax.ShapeDtypeStruct((n, dh), F32)] * 2
    res = pl.pallas_call(
        body, grid=(n // blk,),
        in_specs=[
            pl.BlockSpec((blk, din), lambda i: (i, 0)),
            pl.BlockSpec((blk, dout), lambda i: (i, 0)),
            pl.BlockSpec((din, dout), lambda i: (0, 0)),
            pl.BlockSpec((1, dout), lambda i: (0, 0)),
            pl.BlockSpec((1, dout), lambda i: (0, 0)),
            pl.BlockSpec((1, dout), lambda i: (0, 0)),
        ],
        out_specs=out_specs if halves else out_specs[0],
        out_shape=out_shape if halves else out_shape[0],
    )(y, idn, w, _r2(b), _r2(g), _r2(be))
    return res


def _tc_dinv(degp, n_pad):
    f = n_pad * 16
    x = degp.reshape(2, f)
    blk = 6400 if f % 6400 == 0 else 4096

    def body(p_r, o_r):
        xx = p_r[...]
        o_r[...] = lax.rsqrt(1.0 + xx[0:1, :] + xx[1:2, :])

    out = pl.pallas_call(
        body, grid=(f // blk,),
        in_specs=[pl.BlockSpec((2, blk), lambda i: (0, i))],
        out_specs=pl.BlockSpec((1, blk), lambda i: (0, i)),
        out_shape=jax.ShapeDtypeStruct((1, f), F32),
    )(x)
    return out.reshape(n_pad, 16)


def _tc_head(x, w, b):
    n, din = x.shape
    dout = w.shape[1]
    blk = _bs(n)

    def body(x_r, w_r, b_r, o_r):
        o_r[...] = jnp.dot(x_r[...], w_r[...],
                           preferred_element_type=F32) + b_r[...]

    return pl.pallas_call(
        body, grid=(n // blk,),
        in_specs=[
            pl.BlockSpec((blk, din), lambda i: (i, 0)),
            pl.BlockSpec((din, dout), lambda i: (0, 0)),
            pl.BlockSpec((1, dout), lambda i: (0, 0)),
        ],
        out_specs=pl.BlockSpec((blk, dout), lambda i: (i, 0)),
        out_shape=jax.ShapeDtypeStruct((n, dout), F32),
    )(x, w, _r2(b))


def _tc_head2(x, w1, b1, w2, b2):
    n, din = x.shape
    d1, d2 = w1.shape[1], w2.shape[1]
    blk = _bs(n)

    def body(x_r, w1_r, b1_r, w2_r, b2_r, o1_r, o2_r):
        xx = x_r[...]
        o1_r[...] = jnp.dot(xx, w1_r[...], preferred_element_type=F32) + b1_r[...]
        o2_r[...] = jnp.dot(xx, w2_r[...], preferred_element_type=F32) + b2_r[...]

    return pl.pallas_call(
        body, grid=(n // blk,),
        in_specs=[
            pl.BlockSpec((blk, din), lambda i: (i, 0)),
            pl.BlockSpec((din, d1), lambda i: (0, 0)),
            pl.BlockSpec((1, d1), lambda i: (0, 0)),
            pl.BlockSpec((din, d2), lambda i: (0, 0)),
            pl.BlockSpec((1, d2), lambda i: (0, 0)),
        ],
        out_specs=[
            pl.BlockSpec((blk, d1), lambda i: (i, 0)),
            pl.BlockSpec((blk, d2), lambda i: (i, 0)),
        ],
        out_shape=[
            jax.ShapeDtypeStruct((n, d1), F32),
            jax.ShapeDtypeStruct((n, d2), F32),
        ],
    )(x, w1, _r2(b1), w2, _r2(b2))


def _conv_block(x, skip, rowp, colp, normp, dinv2, p, pre, n_pad, d, fine,
                add_idn_last, idn):
    out = x
    for i in range(2):
        w = p[pre + '_convW' + str(i)]
        b = p[pre + '_convb' + str(i)]
        ng = p[pre + '_ng' + str(i)]
        nb = p[pre + '_nb' + str(i)]
        sk = skip if i == 0 else None
        last = add_idn_last and i == 1
        if fine:
            hcat = _tc_mm(out, w, skip=sk, halves=True)
            oflat = _sc_scatter_split(hcat, rowp, colp, normp, n_pad,
                                      d // 2, n_pad)
            out = _tc_post([oflat], [hcat], dinv2, b, ng, nb,
                           idn=idn if last else None, split=True)
        else:
            h = _tc_mm(out, w, skip=sk)
            pp = _sc_scatter(h, rowp, colp, normp, n_pad, d)
            out = _tc_post([pp], [h], dinv2, b, ng, nb,
                           idn=idn if last else None)
    return out


def _encoder(x, ed, p, pre, n_pad, d, fine):
    idn = _tc_mlp((x,), p[pre + '_res_W'], p[pre + '_res_b'],
                  p[pre + '_res_g'], p[pre + '_res_beta'], 'plain')
    return _conv_block(x, None, ed['row'], ed['col'], ed['norm'], ed['dinv2'],
                       p, pre, n_pad, d, fine, True, idn)


def _decoder(x, skip, ed, p, pre, n_pad, d, fine, proj, halves=False):
    idn = _tc_mlp((x, skip), p[pre + '_res_W'], p[pre + '_res_b'],
                  p[pre + '_res_g'], p[pre + '_res_beta'], 'skip')
    out = _conv_block(x, skip, ed['row'], ed['col'], ed['norm'], ed['dinv2'],
                      p, pre, n_pad, d, fine, not proj, idn)
    if proj:
        return _tc_proj(out, idn, p[pre + '_p_W'], p[pre + '_p_b'],
                        p[pre + '_p_g'], p[pre + '_p_beta'], halves=halves)
    return out


def _edge_level(ei, ew, n, n_pad):
    row = _pad1(ei[0], 4096)
    col = _pad1(ei[1], 4096)
    w = _pad1(ew, 4096)
    degp = _sc_deg(col, w, n_pad)
    dinv2 = _tc_dinv(degp, n_pad)
    dinv1 = dinv2[:, 0]
    norm = _sc_norm(row, col, w, dinv1)
    return dict(row=row, col=col, norm=norm, dinv2=dinv2, dinv1=dinv1)


def kernel(features, edge_index_fine, edge_weight_fine, edge_index_mid,
           edge_weight_mid, edge_index_coarse, edge_weight_coarse,
           map_f2m_idx, map_f2m_val, map_m2c_idx, map_m2c_val, map_c2m_idx,
           map_c2m_val, map_m2f_idx, map_m2f_val, params):
    p = params

    fused = _tc_fus(features, p['fus_W'], p['fus_b'], p['fus_g'],
                    p['fus_beta'], p['fus_mw'])

    edf = _edge_level(edge_index_fine, edge_weight_fine, NFINE, NFINE)
    edm = _edge_level(edge_index_mid, edge_weight_mid, NMID, NMIDP)
    edc = _edge_level(edge_index_coarse, edge_weight_coarse, NCOARSE, NCOARSEP)

    e1 = _encoder(fused, edf, p, 'enc1', NFINE, 64, True)

    f2m_r = _pad1(map_f2m_idx[1], 4096)
    f2m_c = _pad1(map_f2m_idx[0], 4096)
    f2m_v = _pad1(map_f2m_val, 4096)
    m1p = _sc_scatter(e1, f2m_r, f2m_c, f2m_v, NMIDP, 64)
    m1 = _tc_mlp((m1p,), p['f2m_W'], p['f2m_b'], p['f2m_g'], p['f2m_beta'],
                 'part')

    e2 = _encoder(m1, edm, p, 'enc2', NMIDP, 128, False)

    m2c_r = _pad1(map_m2c_idx[1], 4096)
    m2c_c = _pad1(map_m2c_idx[0], 4096)
    m2c_v = _pad1(map_m2c_val, 4096)
    m2p = _sc_scatter(e2, m2c_r, m2c_c, m2c_v, NCOARSEP, 128)
    m2 = _tc_mlp((m2p,), p['m2c_W'], p['m2c_b'], p['m2c_g'], p['m2c_beta'],
                 'part')

    e3 = _encoder(m2, edc, p, 'enc3', NCOARSEP, 256, False)

    d3 = _decoder(e3, m2, edc, p, 'dec3', NCOARSEP, 256, False, True)

    c2m_r = _pad1(map_c2m_idx[1], 4096)
    c2m_c = _pad1(map_c2m_idx[0], 4096)
    c2m_v = _pad1(map_c2m_val, 4096)
    u2p = _sc_scatter(d3, c2m_r, c2m_c, c2m_v, NMIDP, 128)
    u2 = _tc_mlp((u2p,), p['c2m_W'], p['c2m_b'], p['c2m_g'], p['c2m_beta'],
                 'part')

    d2, d2a, d2b = _decoder(u2, e2, edm, p, 'dec2', NMIDP, 128, False, True,
                            halves=True)

    m2f_r = _pad1(map_m2f_idx[1], 4096)
    m2f_c = _pad1(map_m2f_idx[0], 4096)
    m2f_v = _pad1(map_m2f_val, 4096)
    d2h = jnp.concatenate([d2a, d2b], axis=0)
    u1flat = _sc_scatter_split(d2h, m2f_r, m2f_c, m2f_v, NFINE, 32, NMIDP)
    u1 = _tc_mlp((u1flat,), p['m2f_W'], p['m2f_b'], p['m2f_g'],
                 p['m2f_beta'], 'cat2')

    d1 = _decoder(u1, e1, edf, p, 'dec1', NFINE, 64, True, False)

    emb_f, recon = _tc_head2(d1, p['head_f_W'], p['head_f_b'], p['rec_W'],
                             p['rec_b'])
    emb_m = _tc_head(d2, p['head_m_W'], p['head_m_b'])[:NMID]
    emb_c = _tc_head(d3, p['head_c_W'], p['head_c_b'])[:NCOARSE]
    return emb_f, emb_m, emb_c, recon
